# baseline (speedup 1.0000x reference)
import functools

import jax
import jax.numpy as jnp
from jax import lax
from jax.experimental import pallas as pl
from jax.experimental.pallas import tpu as pltpu
from jax.experimental.pallas import tpu_sc as plsc

N = 10000
E = 320000
D = 128
EA = 16
H = 128

NC, NS = 2, 16
NW = NC * NS
EPW = E // NW
C = 80
NCHUNK = EPW // C
RPT = 632

@functools.cache
def _get_mesh():
    return plsc.VectorSubcoreMesh(
        core_axis_name="c", subcore_axis_name="s", num_cores=NC, num_subcores=NS)


@functools.cache
def _make_sc_gather():
    return functools.partial(
        pl.kernel,
        out_type=(
            jax.ShapeDtypeStruct((E, D), jnp.bfloat16),
            jax.ShapeDtypeStruct((E, D), jnp.bfloat16),
            jax.ShapeDtypeStruct((E, 16), jnp.float32),
        ),
        mesh=_get_mesh(),
        scratch_types=[
            pltpu.VMEM((NCHUNK, C), jnp.int32),
            pltpu.VMEM((NCHUNK, C), jnp.int32),
            pltpu.VMEM((C, D), jnp.bfloat16),
            pltpu.VMEM((C, D), jnp.bfloat16),
            pltpu.VMEM((C, 16), jnp.float32),
            pltpu.VMEM((C, 16), jnp.float32),
            pltpu.VMEM((C, D), jnp.bfloat16),
            pltpu.VMEM((C, D), jnp.bfloat16),
            pltpu.VMEM((C, 16), jnp.float32),
            pltpu.VMEM((C, 16), jnp.float32),
            pltpu.SemaphoreType.DMA,
            pltpu.SemaphoreType.DMA,
            pltpu.SemaphoreType.DMA,
            pltpu.SemaphoreType.DMA,
        ],
        compiler_params=pltpu.CompilerParams(use_tc_tiling_on_sc=False),
    )(_sc_gather_body)


def _sc_gather_body(ns_hbm, coordp_hbm, row3_hbm, col3_hbm,
                    src_hbm, dst_hbm, cdiff_hbm,
                    idx_r2, idx_c2,
                    src_v0, dst_v0, cr_v0, cc_v0,
                    src_v1, dst_v1, cr_v1, cc_v1,
                    gsem0, gsem1, wsem0, wsem1):
    wid = lax.axis_index("s") * NC + lax.axis_index("c")
    data = ((src_v0, dst_v0, cr_v0, cc_v0), (src_v1, dst_v1, cr_v1, cc_v1))
    gsem = (gsem0, gsem1)
    wsem = (wsem0, wsem1)

    pltpu.sync_copy(row3_hbm.at[wid], idx_r2)
    pltpu.sync_copy(col3_hbm.at[wid], idx_c2)

    def fire(i, b):
        srcb, dstb, crb, ccb = data[b]
        pltpu.async_copy(ns_hbm.at[idx_r2.at[i]], srcb, gsem[b])
        pltpu.async_copy(ns_hbm.at[idx_c2.at[i]], dstb, gsem[b])
        pltpu.async_copy(coordp_hbm.at[idx_r2.at[i]], crb, gsem[b])
        pltpu.async_copy(coordp_hbm.at[idx_c2.at[i]], ccb, gsem[b])

    def wait_gathers(i, b):
        srcb, dstb, crb, ccb = data[b]
        pltpu.make_async_copy(ns_hbm.at[idx_r2.at[i]], srcb, gsem[b]).wait()
        pltpu.make_async_copy(ns_hbm.at[idx_c2.at[i]], dstb, gsem[b]).wait()
        pltpu.make_async_copy(coordp_hbm.at[idx_r2.at[i]], crb, gsem[b]).wait()
        pltpu.make_async_copy(coordp_hbm.at[idx_c2.at[i]], ccb, gsem[b]).wait()

    def fire_writes(i, b):
        srcb, dstb, crb, _ = data[b]
        base = wid * EPW + i * C
        pltpu.async_copy(srcb, src_hbm.at[pl.ds(base, C)], wsem[b])
        pltpu.async_copy(dstb, dst_hbm.at[pl.ds(base, C)], wsem[b])
        pltpu.async_copy(crb, cdiff_hbm.at[pl.ds(base, C)], wsem[b])

    def drain_writes(i, b):
        srcb, dstb, crb, _ = data[b]
        base = wid * EPW + i * C
        pltpu.make_async_copy(srcb, src_hbm.at[pl.ds(base, C)], wsem[b]).wait()
        pltpu.make_async_copy(dstb, dst_hbm.at[pl.ds(base, C)], wsem[b]).wait()
        pltpu.make_async_copy(crb, cdiff_hbm.at[pl.ds(base, C)], wsem[b]).wait()

    def compute(b):
        _, _, crb, ccb = data[b]

        def sub(j, c2):
            crb[j] = crb[j] - ccb[j]
            return c2
        lax.fori_loop(0, C, sub, 0)

    def stage(i, b):
        @pl.when(i + 1 < NCHUNK)
        def _():
            @pl.when(i >= 1)
            def __():
                drain_writes(i, 1 - b)
            fire(i + 1, 1 - b)
        wait_gathers(i, b)
        compute(b)
        fire_writes(i, b)

    fire(0, 0)

    def outer(g, carry):
        stage(2 * g, 0)
        stage(2 * g + 1, 1)
        return carry
    lax.fori_loop(0, NCHUNK // 2, outer, 0)
    stage(NCHUNK - 1, 0)
    drain_writes(NCHUNK - 1, 0)
    drain_writes(NCHUNK - 2, 1)


@functools.cache
def _make_sc_scatter():
    return functools.partial(
        pl.kernel,
        out_type=(
            jax.ShapeDtypeStruct((NC, N, H), jnp.float32),
            jax.ShapeDtypeStruct((NC, N, 16), jnp.float32),
        ),
        mesh=_get_mesh(),
        scratch_types=[
            pltpu.VMEM((NCHUNK, C), jnp.int32),
            pltpu.VMEM((C, H), jnp.float32),
            pltpu.VMEM((C, 16), jnp.float32),
            pltpu.VMEM((C, H), jnp.float32),
            pltpu.VMEM((C, 16), jnp.float32),
            pltpu.VMEM_SHARED((N, H), jnp.float32),
            pltpu.VMEM_SHARED((N, 16), jnp.float32),
            pltpu.SemaphoreType.DMA,
            pltpu.SemaphoreType.DMA,
        ],
        compiler_params=pltpu.CompilerParams(use_tc_tiling_on_sc=False),
    )(_sc_scatter_body)


def _sc_scatter_body(row3_hbm, ef_hbm, tp_hbm,
                     outf_hbm, outt_hbm,
                     idx2, ef_v0, tp_v0, ef_v1, tp_v1, accf, acct,
                     lsem0, lsem1):
    cid = lax.axis_index("c")
    sid = lax.axis_index("s")
    wid = sid * NC + cid
    efd = (ef_v0, ef_v1)
    tpd = (tp_v0, tp_v1)
    lsem = (lsem0, lsem1)

    pltpu.sync_copy(row3_hbm.at[wid], idx2)

    def zrow(j, carry):
        def zcol(k, c2):
            ef_v0[j, pl.ds(k * 16, 16)] = jnp.zeros((16,), jnp.float32)
            return c2
        lax.fori_loop(0, H // 16, zcol, 0)
        tp_v0[j] = jnp.zeros((16,), jnp.float32)
        return carry
    lax.fori_loop(0, C, zrow, 0)

    def zchunk(c, carry):
        base = pl.multiple_of(jnp.minimum(sid * 640 + c * C, N - C), 8)
        pltpu.sync_copy(ef_v0, accf.at[pl.ds(base, C)])
        pltpu.sync_copy(tp_v0, acct.at[pl.ds(base, C)])
        return carry
    lax.fori_loop(0, 8, zchunk, 0)
    plsc.subcore_barrier()

    def fire_loads(i, b):
        base = wid * EPW + i * C
        pltpu.async_copy(ef_hbm.at[pl.ds(base, C)], efd[b], lsem[b])
        pltpu.async_copy(tp_hbm.at[pl.ds(base, C)], tpd[b], lsem[b])

    def wait_loads(i, b):
        base = wid * EPW + i * C
        pltpu.make_async_copy(ef_hbm.at[pl.ds(base, C)], efd[b], lsem[b]).wait()
        pltpu.make_async_copy(tp_hbm.at[pl.ds(base, C)], tpd[b], lsem[b]).wait()

    def stage(i, b):
        @pl.when(i + 1 < NCHUNK)
        def _():
            fire_loads(i + 1, 1 - b)
        wait_loads(i, b)
        pltpu.sync_copy(efd[b], accf.at[idx2.at[i]], add=True)
        pltpu.sync_copy(tpd[b], acct.at[idx2.at[i]], add=True)

    fire_loads(0, 0)

    def outer(g, carry):
        stage(2 * g, 0)
        stage(2 * g + 1, 1)
        return carry
    lax.fori_loop(0, NCHUNK // 2, outer, 0)
    stage(NCHUNK - 1, 0)
    plsc.subcore_barrier()

    def wchunk(c, carry):
        base = pl.multiple_of(jnp.minimum(sid * 640 + c * C, N - C), 8)
        pltpu.sync_copy(accf.at[pl.ds(base, C)], ef_v0)
        pltpu.sync_copy(ef_v0, outf_hbm.at[cid, pl.ds(base, C)])
        pltpu.sync_copy(acct.at[pl.ds(base, C)], tp_v0)
        pltpu.sync_copy(tp_v0, outt_hbm.at[cid, pl.ds(base, C)])
        return carry
    lax.fori_loop(0, 8, wchunk, 0)


BE = 1000


def _silu(x):
    return x * (1.0 / (1.0 + jnp.exp(-x)))


def _tc_edge_body(src_ref, dst_ref, cd_ref, ea_ref,
                  wa_ref, wb_ref, wr_ref, we_ref, be1_ref,
                  we2_ref, be2_ref, wc1_ref, bc1_ref, wc2t_ref,
                  ef_ref, tp_ref):
    src = src_ref[...]
    dst = dst_ref[...]
    cd = cd_ref[...]
    ea = ea_ref[...]
    rad = jnp.sum(cd * cd, axis=1, keepdims=True)
    pre = (jnp.dot(src, wa_ref[...], preferred_element_type=jnp.float32)
           + jnp.dot(dst, wb_ref[...], preferred_element_type=jnp.float32)
           + jnp.dot(ea, we_ref[...], preferred_element_type=jnp.float32)
           + rad * wr_ref[...]
           + be1_ref[...])
    h = _silu(pre)
    ef = _silu(jnp.dot(h, we2_ref[...], preferred_element_type=jnp.float32) + be2_ref[...])
    ef_ref[...] = ef
    hc = _silu(jnp.dot(ef, wc1_ref[...], preferred_element_type=jnp.float32) + bc1_ref[...])
    cm = jnp.sum(hc * wc2t_ref[...], axis=1, keepdims=True)
    trans = jnp.clip(cm * cd, -100.0, 100.0)
    ones_col = (lax.broadcasted_iota(jnp.int32, (BE, 16), 1) == 3).astype(jnp.float32)
    tp_ref[...] = trans + ones_col


def _tc_edge(src, dst, cdiff, edge_attr, wa, wb, wr, wee, be1, we2, be2, wc1, bc1, wc2t):
    grid = (E // BE,)
    full = lambda i: (0, 0)
    return pl.pallas_call(
        _tc_edge_body,
        grid=grid,
        in_specs=[
            pl.BlockSpec((BE, D), lambda i: (i, 0)),
            pl.BlockSpec((BE, D), lambda i: (i, 0)),
            pl.BlockSpec((BE, 16), lambda i: (i, 0)),
            pl.BlockSpec((BE, EA), lambda i: (i, 0)),
            pl.BlockSpec((D, H), full),
            pl.BlockSpec((D, H), full),
            pl.BlockSpec((1, H), full),
            pl.BlockSpec((EA, H), full),
            pl.BlockSpec((1, H), full),
            pl.BlockSpec((H, H), full),
            pl.BlockSpec((1, H), full),
            pl.BlockSpec((H, H), full),
            pl.BlockSpec((1, H), full),
            pl.BlockSpec((1, H), full),
        ],
        out_specs=[
            pl.BlockSpec((BE, H), lambda i: (i, 0)),
            pl.BlockSpec((BE, 16), lambda i: (i, 0)),
        ],
        out_shape=[
            jax.ShapeDtypeStruct((E, H), jnp.float32),
            jax.ShapeDtypeStruct((E, 16), jnp.float32),
        ],
    )(src, dst, cdiff, edge_attr, wa, wb, wr, wee, be1, we2, be2, wc1, bc1, wc2t)


BN = 2000


def _tc_node_body(ns_ref, cp_ref, vp_ref, sf_ref, st_ref,
                  wn1a_ref, wn1b_ref, bn1_ref, wn2_ref, bn2_ref,
                  wv1_ref, bv1_ref, wv2t_ref, bv2_ref,
                  nsout_ref, c3_ref):
    ns = ns_ref[...]
    sf = sf_ref[0] + sf_ref[1]
    st = st_ref[0] + st_ref[1]
    cnt = jnp.maximum(st[:, 3:4], 1.0)
    agg_f = sf / cnt
    agg_t = st / cnt
    pre = (jnp.dot(ns, wn1a_ref[...], preferred_element_type=jnp.float32)
           + jnp.dot(agg_f, wn1b_ref[...], preferred_element_type=jnp.float32)
           + bn1_ref[...])
    h = _silu(pre)
    nout = jnp.dot(h, wn2_ref[...], preferred_element_type=jnp.float32) + bn2_ref[...]
    nsout_ref[...] = ns + nout
    pv = jnp.dot(ns, wv1_ref[...], preferred_element_type=jnp.float32) + bv1_ref[...]
    hv = _silu(pv)
    cvm = jnp.sum(hv * wv2t_ref[...], axis=1, keepdims=True) + bv2_ref[...]
    mask3 = (lax.broadcasted_iota(jnp.int32, (BN, 16), 1) < 3).astype(jnp.float32)
    c3_ref[...] = cp_ref[...] + agg_t * mask3 + cvm * vp_ref[...]


def _tc_node(ns, coordp, velp, sumf, sumt, wn1a, wn1b, bn1, wn2, bn2, wv1, bv1, wv2t, bv2):
    grid = (N // BN,)
    full = lambda i: (0, 0)
    return pl.pallas_call(
        _tc_node_body,
        grid=grid,
        in_specs=[
            pl.BlockSpec((BN, D), lambda i: (i, 0)),
            pl.BlockSpec((BN, 16), lambda i: (i, 0)),
            pl.BlockSpec((BN, 16), lambda i: (i, 0)),
            pl.BlockSpec((NC, BN, H), lambda i: (0, i, 0)),
            pl.BlockSpec((NC, BN, 16), lambda i: (0, i, 0)),
            pl.BlockSpec((D, H), full),
            pl.BlockSpec((H, H), full),
            pl.BlockSpec((1, H), full),
            pl.BlockSpec((H, D), full),
            pl.BlockSpec((1, D), full),
            pl.BlockSpec((D, H), full),
            pl.BlockSpec((1, H), full),
            pl.BlockSpec((1, H), full),
            pl.BlockSpec((1, 1), full),
        ],
        out_specs=[
            pl.BlockSpec((BN, D), lambda i: (i, 0)),
            pl.BlockSpec((BN, 16), lambda i: (i, 0)),
        ],
        out_shape=[
            jax.ShapeDtypeStruct((N, D), jnp.float32),
            jax.ShapeDtypeStruct((N, 16), jnp.float32),
        ],
    )(ns, coordp, velp, sumf, sumt, wn1a, wn1b, bn1, wn2, bn2, wv1, bv1, wv2t, bv2)


def kernel(node_state, edge_index, coord, velocity, edge_attr,
           We1, be1, We2, be2, Wn1, bn1, Wn2, bn2,
           Wc1, bc1, Wc2, Wv1, bv1, Wv2, bv2):
    row = edge_index[0]
    col = edge_index[1]
    row3 = row.reshape(NW, NCHUNK, C)
    col3 = col.reshape(NW, NCHUNK, C)
    coordp = jnp.pad(coord, ((0, 0), (0, 13)))
    velp = jnp.pad(velocity, ((0, 0), (0, 13)))

    ns_bf = node_state.astype(jnp.bfloat16)
    src, dst, cdiff = _make_sc_gather()(ns_bf, coordp, row3, col3)

    wa = We1[:D].astype(jnp.bfloat16)
    wb = We1[D:2 * D].astype(jnp.bfloat16)
    wr = We1[2 * D:2 * D + 1]
    wee = We1[2 * D + 1:]
    ef, tp = _tc_edge(src, dst, cdiff, edge_attr,
                      wa, wb, wr, wee, be1.reshape(1, H),
                      We2, be2.reshape(1, H), Wc1, bc1.reshape(1, H),
                      Wc2.reshape(1, H))

    sumf, sumt = _make_sc_scatter()(row3, ef, tp)

    ns_new, c3p = _tc_node(node_state, coordp, velp, sumf, sumt,
                           Wn1[:D], Wn1[D:], bn1.reshape(1, H),
                           Wn2, bn2.reshape(1, D),
                           Wv1, bv1.reshape(1, H), Wv2.reshape(1, H),
                           bv2.reshape(1, 1))

    c3 = c3p[:, :3].reshape(N, 3, 1)
    v3 = velocity.reshape(N, 3, 1)
    return ns_new, c3, v3


# speedup vs baseline: 1.0233x; 1.0233x over previous
import functools

import jax
import jax.numpy as jnp
from jax import lax
from jax.experimental import pallas as pl
from jax.experimental.pallas import tpu as pltpu
from jax.experimental.pallas import tpu_sc as plsc

N = 10000
E = 320000
D = 128
EA = 16
H = 128

NC, NS = 2, 16
NW = NC * NS
EPW = E // NW
C = 80
NCHUNK = EPW // C
RPT = 632

@functools.cache
def _get_mesh():
    return plsc.VectorSubcoreMesh(
        core_axis_name="c", subcore_axis_name="s", num_cores=NC, num_subcores=NS)


@functools.cache
def _make_sc_gather():
    return functools.partial(
        pl.kernel,
        out_type=(
            jax.ShapeDtypeStruct((E, D), jnp.bfloat16),
            jax.ShapeDtypeStruct((E, D), jnp.bfloat16),
            jax.ShapeDtypeStruct((E, 16), jnp.float32),
        ),
        mesh=_get_mesh(),
        scratch_types=[
            pltpu.VMEM((NCHUNK, C), jnp.int32),
            pltpu.VMEM((NCHUNK, C), jnp.int32),
            pltpu.VMEM((C, D), jnp.bfloat16),
            pltpu.VMEM((C, D), jnp.bfloat16),
            pltpu.VMEM((C, 16), jnp.float32),
            pltpu.VMEM((C, 16), jnp.float32),
            pltpu.VMEM((C, D), jnp.bfloat16),
            pltpu.VMEM((C, D), jnp.bfloat16),
            pltpu.VMEM((C, 16), jnp.float32),
            pltpu.VMEM((C, 16), jnp.float32),
            pltpu.SemaphoreType.DMA,
            pltpu.SemaphoreType.DMA,
            pltpu.SemaphoreType.DMA,
            pltpu.SemaphoreType.DMA,
        ],
        compiler_params=pltpu.CompilerParams(use_tc_tiling_on_sc=False),
    )(_sc_gather_body)


def _sc_gather_body(ns_hbm, coordp_hbm, row3_hbm, col3_hbm,
                    src_hbm, dst_hbm, cdiff_hbm,
                    idx_r2, idx_c2,
                    src_v0, dst_v0, cr_v0, cc_v0,
                    src_v1, dst_v1, cr_v1, cc_v1,
                    gsem0, gsem1, wsem0, wsem1):
    wid = lax.axis_index("s") * NC + lax.axis_index("c")
    data = ((src_v0, dst_v0, cr_v0, cc_v0), (src_v1, dst_v1, cr_v1, cc_v1))
    gsem = (gsem0, gsem1)
    wsem = (wsem0, wsem1)

    pltpu.sync_copy(row3_hbm.at[wid], idx_r2)
    pltpu.sync_copy(col3_hbm.at[wid], idx_c2)

    def fire(i, b):
        srcb, dstb, crb, ccb = data[b]
        pltpu.async_copy(ns_hbm.at[idx_r2.at[i]], srcb, gsem[b])
        pltpu.async_copy(ns_hbm.at[idx_c2.at[i]], dstb, gsem[b])
        pltpu.async_copy(coordp_hbm.at[idx_r2.at[i]], crb, gsem[b])
        pltpu.async_copy(coordp_hbm.at[idx_c2.at[i]], ccb, gsem[b])

    def wait_gathers(i, b):
        srcb, dstb, crb, ccb = data[b]
        pltpu.make_async_copy(ns_hbm.at[idx_r2.at[i]], srcb, gsem[b]).wait()
        pltpu.make_async_copy(ns_hbm.at[idx_c2.at[i]], dstb, gsem[b]).wait()
        pltpu.make_async_copy(coordp_hbm.at[idx_r2.at[i]], crb, gsem[b]).wait()
        pltpu.make_async_copy(coordp_hbm.at[idx_c2.at[i]], ccb, gsem[b]).wait()

    def fire_writes(i, b):
        srcb, dstb, crb, _ = data[b]
        base = wid * EPW + i * C
        pltpu.async_copy(srcb, src_hbm.at[pl.ds(base, C)], wsem[b])
        pltpu.async_copy(dstb, dst_hbm.at[pl.ds(base, C)], wsem[b])
        pltpu.async_copy(crb, cdiff_hbm.at[pl.ds(base, C)], wsem[b])

    def drain_writes(i, b):
        srcb, dstb, crb, _ = data[b]
        base = wid * EPW + i * C
        pltpu.make_async_copy(srcb, src_hbm.at[pl.ds(base, C)], wsem[b]).wait()
        pltpu.make_async_copy(dstb, dst_hbm.at[pl.ds(base, C)], wsem[b]).wait()
        pltpu.make_async_copy(crb, cdiff_hbm.at[pl.ds(base, C)], wsem[b]).wait()

    def compute(b):
        _, _, crb, ccb = data[b]

        def sub(j, c2):
            crb[j] = crb[j] - ccb[j]
            return c2
        lax.fori_loop(0, C, sub, 0)

    def stage(i, b):
        @pl.when(i + 1 < NCHUNK)
        def _():
            @pl.when(i >= 1)
            def __():
                drain_writes(i, 1 - b)
            fire(i + 1, 1 - b)
        wait_gathers(i, b)
        compute(b)
        fire_writes(i, b)

    fire(0, 0)

    def outer(g, carry):
        stage(2 * g, 0)
        stage(2 * g + 1, 1)
        return carry
    lax.fori_loop(0, NCHUNK // 2, outer, 0)
    stage(NCHUNK - 1, 0)
    drain_writes(NCHUNK - 1, 0)
    drain_writes(NCHUNK - 2, 1)


@functools.cache
def _make_sc_scatter():
    return functools.partial(
        pl.kernel,
        out_type=(
            jax.ShapeDtypeStruct((NC, N, H), jnp.float32),
            jax.ShapeDtypeStruct((NC, N, 16), jnp.float32),
        ),
        mesh=_get_mesh(),
        scratch_types=[
            pltpu.VMEM((NCHUNK, C), jnp.int32),
            pltpu.VMEM((C, H), jnp.float32),
            pltpu.VMEM((C, 16), jnp.float32),
            pltpu.VMEM((C, H), jnp.float32),
            pltpu.VMEM((C, 16), jnp.float32),
            pltpu.VMEM_SHARED((N, H), jnp.float32),
            pltpu.VMEM_SHARED((N, 16), jnp.float32),
            pltpu.SemaphoreType.DMA,
            pltpu.SemaphoreType.DMA,
        ],
        compiler_params=pltpu.CompilerParams(use_tc_tiling_on_sc=False),
    )(_sc_scatter_body)


def _sc_scatter_body(row3_hbm, ef_hbm, tp_hbm,
                     outf_hbm, outt_hbm,
                     idx2, ef_v0, tp_v0, ef_v1, tp_v1, accf, acct,
                     lsem0, lsem1):
    cid = lax.axis_index("c")
    sid = lax.axis_index("s")
    wid = sid * NC + cid
    efd = (ef_v0, ef_v1)
    tpd = (tp_v0, tp_v1)
    lsem = (lsem0, lsem1)

    pltpu.sync_copy(row3_hbm.at[wid], idx2)

    def zrow(j, carry):
        def zcol(k, c2):
            ef_v0[j, pl.ds(k * 16, 16)] = jnp.zeros((16,), jnp.float32)
            return c2
        lax.fori_loop(0, H // 16, zcol, 0)
        tp_v0[j] = jnp.zeros((16,), jnp.float32)
        return carry
    lax.fori_loop(0, C, zrow, 0)

    def zchunk(c, carry):
        base = pl.multiple_of(jnp.minimum(sid * 640 + c * C, N - C), 8)
        pltpu.sync_copy(ef_v0, accf.at[pl.ds(base, C)])
        pltpu.sync_copy(tp_v0, acct.at[pl.ds(base, C)])
        return carry
    lax.fori_loop(0, 8, zchunk, 0)
    plsc.subcore_barrier()

    def fire_loads(i, b):
        base = wid * EPW + i * C
        pltpu.async_copy(ef_hbm.at[pl.ds(base, C)], efd[b], lsem[b])
        pltpu.async_copy(tp_hbm.at[pl.ds(base, C)], tpd[b], lsem[b])

    def wait_loads(i, b):
        base = wid * EPW + i * C
        pltpu.make_async_copy(ef_hbm.at[pl.ds(base, C)], efd[b], lsem[b]).wait()
        pltpu.make_async_copy(tp_hbm.at[pl.ds(base, C)], tpd[b], lsem[b]).wait()

    def stage(i, b):
        @pl.when(i + 1 < NCHUNK)
        def _():
            fire_loads(i + 1, 1 - b)
        wait_loads(i, b)
        pltpu.sync_copy(efd[b], accf.at[idx2.at[i]], add=True)
        pltpu.sync_copy(tpd[b], acct.at[idx2.at[i]], add=True)

    fire_loads(0, 0)

    def outer(g, carry):
        stage(2 * g, 0)
        stage(2 * g + 1, 1)
        return carry
    lax.fori_loop(0, NCHUNK // 2, outer, 0)
    stage(NCHUNK - 1, 0)
    plsc.subcore_barrier()

    def wchunk(c, carry):
        base = pl.multiple_of(jnp.minimum(sid * 640 + c * C, N - C), 8)
        pltpu.sync_copy(accf.at[pl.ds(base, C)], ef_v0)
        pltpu.sync_copy(ef_v0, outf_hbm.at[cid, pl.ds(base, C)])
        pltpu.sync_copy(acct.at[pl.ds(base, C)], tp_v0)
        pltpu.sync_copy(tp_v0, outt_hbm.at[cid, pl.ds(base, C)])
        return carry
    lax.fori_loop(0, 8, wchunk, 0)


BE = 1000


def _silu(x):
    return x * (1.0 / (1.0 + jnp.exp(-x)))


def _tc_edge_body(src_ref, dst_ref, cd_ref, ea_ref,
                  wa_ref, wb_ref, wr_ref, we_ref, be1_ref,
                  we2_ref, be2_ref, wc1_ref, bc1_ref, wc2t_ref,
                  ef_ref, tp_ref):
    src = src_ref[...].astype(jnp.float32)
    dst = dst_ref[...].astype(jnp.float32)
    cd = cd_ref[...]
    ea = ea_ref[...]
    rad = jnp.sum(cd * cd, axis=1, keepdims=True)
    pre = (jnp.dot(src, wa_ref[...], preferred_element_type=jnp.float32)
           + jnp.dot(dst, wb_ref[...], preferred_element_type=jnp.float32)
           + jnp.dot(ea, we_ref[...], preferred_element_type=jnp.float32)
           + rad * wr_ref[...]
           + be1_ref[...])
    h = _silu(pre)
    ef = _silu(jnp.dot(h, we2_ref[...], preferred_element_type=jnp.float32) + be2_ref[...])
    ef_ref[...] = ef
    hc = _silu(jnp.dot(ef, wc1_ref[...], preferred_element_type=jnp.float32) + bc1_ref[...])
    cm = jnp.sum(hc * wc2t_ref[...], axis=1, keepdims=True)
    trans = jnp.clip(cm * cd, -100.0, 100.0)
    ones_col = (lax.broadcasted_iota(jnp.int32, (BE, 16), 1) == 3).astype(jnp.float32)
    tp_ref[...] = trans + ones_col


def _tc_edge(src, dst, cdiff, edge_attr, wa, wb, wr, wee, be1, we2, be2, wc1, bc1, wc2t):
    grid = (E // BE,)
    full = lambda i: (0, 0)
    return pl.pallas_call(
        _tc_edge_body,
        grid=grid,
        in_specs=[
            pl.BlockSpec((BE, D), lambda i: (i, 0)),
            pl.BlockSpec((BE, D), lambda i: (i, 0)),
            pl.BlockSpec((BE, 16), lambda i: (i, 0)),
            pl.BlockSpec((BE, EA), lambda i: (i, 0)),
            pl.BlockSpec((D, H), full),
            pl.BlockSpec((D, H), full),
            pl.BlockSpec((1, H), full),
            pl.BlockSpec((EA, H), full),
            pl.BlockSpec((1, H), full),
            pl.BlockSpec((H, H), full),
            pl.BlockSpec((1, H), full),
            pl.BlockSpec((H, H), full),
            pl.BlockSpec((1, H), full),
            pl.BlockSpec((1, H), full),
        ],
        out_specs=[
            pl.BlockSpec((BE, H), lambda i: (i, 0)),
            pl.BlockSpec((BE, 16), lambda i: (i, 0)),
        ],
        out_shape=[
            jax.ShapeDtypeStruct((E, H), jnp.float32),
            jax.ShapeDtypeStruct((E, 16), jnp.float32),
        ],
    )(src, dst, cdiff, edge_attr, wa, wb, wr, wee, be1, we2, be2, wc1, bc1, wc2t)


BN = 2000


def _tc_node_body(ns_ref, cp_ref, vp_ref, sf_ref, st_ref,
                  wn1a_ref, wn1b_ref, bn1_ref, wn2_ref, bn2_ref,
                  wv1_ref, bv1_ref, wv2t_ref, bv2_ref,
                  nsout_ref, c3_ref):
    ns = ns_ref[...]
    sf = sf_ref[0] + sf_ref[1]
    st = st_ref[0] + st_ref[1]
    cnt = jnp.maximum(st[:, 3:4], 1.0)
    agg_f = sf / cnt
    agg_t = st / cnt
    pre = (jnp.dot(ns, wn1a_ref[...], preferred_element_type=jnp.float32)
           + jnp.dot(agg_f, wn1b_ref[...], preferred_element_type=jnp.float32)
           + bn1_ref[...])
    h = _silu(pre)
    nout = jnp.dot(h, wn2_ref[...], preferred_element_type=jnp.float32) + bn2_ref[...]
    nsout_ref[...] = ns + nout
    pv = jnp.dot(ns, wv1_ref[...], preferred_element_type=jnp.float32) + bv1_ref[...]
    hv = _silu(pv)
    cvm = jnp.sum(hv * wv2t_ref[...], axis=1, keepdims=True) + bv2_ref[...]
    mask3 = (lax.broadcasted_iota(jnp.int32, (BN, 16), 1) < 3).astype(jnp.float32)
    c3_ref[...] = cp_ref[...] + agg_t * mask3 + cvm * vp_ref[...]


def _tc_node(ns, coordp, velp, sumf, sumt, wn1a, wn1b, bn1, wn2, bn2, wv1, bv1, wv2t, bv2):
    grid = (N // BN,)
    full = lambda i: (0, 0)
    return pl.pallas_call(
        _tc_node_body,
        grid=grid,
        in_specs=[
            pl.BlockSpec((BN, D), lambda i: (i, 0)),
            pl.BlockSpec((BN, 16), lambda i: (i, 0)),
            pl.BlockSpec((BN, 16), lambda i: (i, 0)),
            pl.BlockSpec((NC, BN, H), lambda i: (0, i, 0)),
            pl.BlockSpec((NC, BN, 16), lambda i: (0, i, 0)),
            pl.BlockSpec((D, H), full),
            pl.BlockSpec((H, H), full),
            pl.BlockSpec((1, H), full),
            pl.BlockSpec((H, D), full),
            pl.BlockSpec((1, D), full),
            pl.BlockSpec((D, H), full),
            pl.BlockSpec((1, H), full),
            pl.BlockSpec((1, H), full),
            pl.BlockSpec((1, 1), full),
        ],
        out_specs=[
            pl.BlockSpec((BN, D), lambda i: (i, 0)),
            pl.BlockSpec((BN, 16), lambda i: (i, 0)),
        ],
        out_shape=[
            jax.ShapeDtypeStruct((N, D), jnp.float32),
            jax.ShapeDtypeStruct((N, 16), jnp.float32),
        ],
    )(ns, coordp, velp, sumf, sumt, wn1a, wn1b, bn1, wn2, bn2, wv1, bv1, wv2t, bv2)


def kernel(node_state, edge_index, coord, velocity, edge_attr,
           We1, be1, We2, be2, Wn1, bn1, Wn2, bn2,
           Wc1, bc1, Wc2, Wv1, bv1, Wv2, bv2):
    row = edge_index[0]
    col = edge_index[1]
    row3 = row.reshape(NW, NCHUNK, C)
    col3 = col.reshape(NW, NCHUNK, C)
    coordp = jnp.pad(coord, ((0, 0), (0, 13)))
    velp = jnp.pad(velocity, ((0, 0), (0, 13)))

    ns_bf = node_state.astype(jnp.bfloat16)
    src, dst, cdiff = _make_sc_gather()(ns_bf, coordp, row3, col3)

    wa = We1[:D]
    wb = We1[D:2 * D]
    wr = We1[2 * D:2 * D + 1]
    wee = We1[2 * D + 1:]
    ef, tp = _tc_edge(src, dst, cdiff, edge_attr,
                      wa, wb, wr, wee, be1.reshape(1, H),
                      We2, be2.reshape(1, H), Wc1, bc1.reshape(1, H),
                      Wc2.reshape(1, H))

    sumf, sumt = _make_sc_scatter()(row3, ef, tp)

    ns_new, c3p = _tc_node(node_state, coordp, velp, sumf, sumt,
                           Wn1[:D], Wn1[D:], bn1.reshape(1, H),
                           Wn2, bn2.reshape(1, D),
                           Wv1, bv1.reshape(1, H), Wv2.reshape(1, H),
                           bv2.reshape(1, 1))

    c3 = c3p[:, :3].reshape(N, 3, 1)
    v3 = velocity.reshape(N, 3, 1)
    return ns_new, c3, v3


# speedup vs baseline: 1.3480x; 1.3173x over previous
import functools

import jax
import jax.numpy as jnp
from jax import lax
from jax.experimental import pallas as pl
from jax.experimental.pallas import tpu as pltpu
from jax.experimental.pallas import tpu_sc as plsc

N = 10000
E = 320000
D = 128
EA = 16
H = 128

NC, NS = 2, 16
NW = NC * NS
EPW = E // NW
C = 80
NCHUNK = EPW // C
RPT = 632

@functools.cache
def _get_mesh():
    return plsc.VectorSubcoreMesh(
        core_axis_name="c", subcore_axis_name="s", num_cores=NC, num_subcores=NS)


@functools.cache
def _make_sc_gather():
    return functools.partial(
        pl.kernel,
        out_type=(
            jax.ShapeDtypeStruct((E, D), jnp.float32),
            jax.ShapeDtypeStruct((E, 16), jnp.float32),
        ),
        mesh=_get_mesh(),
        scratch_types=[
            pltpu.VMEM((NCHUNK, C), jnp.int32),
            pltpu.VMEM((NCHUNK, C), jnp.int32),
            pltpu.VMEM((C, D), jnp.float32),
            pltpu.VMEM((C, D), jnp.float32),
            pltpu.VMEM((C, 16), jnp.float32),
            pltpu.VMEM((C, 16), jnp.float32),
            pltpu.VMEM((C, D), jnp.float32),
            pltpu.VMEM((C, D), jnp.float32),
            pltpu.VMEM((C, 16), jnp.float32),
            pltpu.VMEM((C, 16), jnp.float32),
            pltpu.SemaphoreType.DMA,
            pltpu.SemaphoreType.DMA,
            pltpu.SemaphoreType.DMA,
            pltpu.SemaphoreType.DMA,
        ],
        compiler_params=pltpu.CompilerParams(use_tc_tiling_on_sc=False),
    )(_sc_gather_body)


def _sc_gather_body(p_hbm, q_hbm, coordp_hbm, row3_hbm, col3_hbm,
                    hpq_hbm, cdiff_hbm,
                    idx_r2, idx_c2,
                    src_v0, dst_v0, cr_v0, cc_v0,
                    src_v1, dst_v1, cr_v1, cc_v1,
                    gsem0, gsem1, wsem0, wsem1):
    wid = lax.axis_index("s") * NC + lax.axis_index("c")
    data = ((src_v0, dst_v0, cr_v0, cc_v0), (src_v1, dst_v1, cr_v1, cc_v1))
    gsem = (gsem0, gsem1)
    wsem = (wsem0, wsem1)

    pltpu.sync_copy(row3_hbm.at[wid], idx_r2)
    pltpu.sync_copy(col3_hbm.at[wid], idx_c2)

    def fire(i, b):
        srcb, dstb, crb, ccb = data[b]
        pltpu.async_copy(p_hbm.at[idx_r2.at[i]], srcb, gsem[b])
        pltpu.async_copy(q_hbm.at[idx_c2.at[i]], dstb, gsem[b])
        pltpu.async_copy(coordp_hbm.at[idx_r2.at[i]], crb, gsem[b])
        pltpu.async_copy(coordp_hbm.at[idx_c2.at[i]], ccb, gsem[b])

    def wait_gathers(i, b):
        srcb, dstb, crb, ccb = data[b]
        pltpu.make_async_copy(p_hbm.at[idx_r2.at[i]], srcb, gsem[b]).wait()
        pltpu.make_async_copy(q_hbm.at[idx_c2.at[i]], dstb, gsem[b]).wait()
        pltpu.make_async_copy(coordp_hbm.at[idx_r2.at[i]], crb, gsem[b]).wait()
        pltpu.make_async_copy(coordp_hbm.at[idx_c2.at[i]], ccb, gsem[b]).wait()

    def fire_writes(i, b):
        srcb, _, crb, _ = data[b]
        base = wid * EPW + i * C
        pltpu.async_copy(srcb, hpq_hbm.at[pl.ds(base, C)], wsem[b])
        pltpu.async_copy(crb, cdiff_hbm.at[pl.ds(base, C)], wsem[b])

    def drain_writes(i, b):
        srcb, _, crb, _ = data[b]
        base = wid * EPW + i * C
        pltpu.make_async_copy(srcb, hpq_hbm.at[pl.ds(base, C)], wsem[b]).wait()
        pltpu.make_async_copy(crb, cdiff_hbm.at[pl.ds(base, C)], wsem[b]).wait()

    def compute(b):
        srcb, dstb, crb, ccb = data[b]

        def rowop(j, c2):
            for k in range(D // 16):
                sl = pl.ds(k * 16, 16)
                srcb[j, sl] = srcb[j, sl] + dstb[j, sl]
            crb[j] = crb[j] - ccb[j]
            return c2
        lax.fori_loop(0, C, rowop, 0)

    def stage(i, b):
        @pl.when(i + 1 < NCHUNK)
        def _():
            @pl.when(i >= 1)
            def __():
                drain_writes(i, 1 - b)
            fire(i + 1, 1 - b)
        wait_gathers(i, b)
        compute(b)
        fire_writes(i, b)

    fire(0, 0)

    def outer(g, carry):
        stage(2 * g, 0)
        stage(2 * g + 1, 1)
        return carry
    lax.fori_loop(0, NCHUNK // 2, outer, 0)
    stage(NCHUNK - 1, 0)
    drain_writes(NCHUNK - 1, 0)
    drain_writes(NCHUNK - 2, 1)


@functools.cache
def _make_sc_scatter():
    return functools.partial(
        pl.kernel,
        out_type=(
            jax.ShapeDtypeStruct((NC, N, H), jnp.float32),
            jax.ShapeDtypeStruct((NC, N, 16), jnp.float32),
        ),
        mesh=_get_mesh(),
        scratch_types=[
            pltpu.VMEM((NCHUNK, C), jnp.int32),
            pltpu.VMEM((C, H), jnp.float32),
            pltpu.VMEM((C, 16), jnp.float32),
            pltpu.VMEM((C, H), jnp.float32),
            pltpu.VMEM((C, 16), jnp.float32),
            pltpu.VMEM_SHARED((N, H), jnp.float32),
            pltpu.VMEM_SHARED((N, 16), jnp.float32),
            pltpu.SemaphoreType.DMA,
            pltpu.SemaphoreType.DMA,
        ],
        compiler_params=pltpu.CompilerParams(use_tc_tiling_on_sc=False),
    )(_sc_scatter_body)


def _sc_scatter_body(row3_hbm, ef_hbm, tp_hbm,
                     outf_hbm, outt_hbm,
                     idx2, ef_v0, tp_v0, ef_v1, tp_v1, accf, acct,
                     lsem0, lsem1):
    cid = lax.axis_index("c")
    sid = lax.axis_index("s")
    wid = sid * NC + cid
    efd = (ef_v0, ef_v1)
    tpd = (tp_v0, tp_v1)
    lsem = (lsem0, lsem1)

    pltpu.sync_copy(row3_hbm.at[wid], idx2)

    def zrow(j, carry):
        def zcol(k, c2):
            ef_v0[j, pl.ds(k * 16, 16)] = jnp.zeros((16,), jnp.float32)
            return c2
        lax.fori_loop(0, H // 16, zcol, 0)
        tp_v0[j] = jnp.zeros((16,), jnp.float32)
        return carry
    lax.fori_loop(0, C, zrow, 0)

    def zchunk(c, carry):
        base = pl.multiple_of(jnp.minimum(sid * 640 + c * C, N - C), 8)
        pltpu.sync_copy(ef_v0, accf.at[pl.ds(base, C)])
        pltpu.sync_copy(tp_v0, acct.at[pl.ds(base, C)])
        return carry
    lax.fori_loop(0, 8, zchunk, 0)
    plsc.subcore_barrier()

    def fire_loads(i, b):
        base = wid * EPW + i * C
        pltpu.async_copy(ef_hbm.at[pl.ds(base, C)], efd[b], lsem[b])
        pltpu.async_copy(tp_hbm.at[pl.ds(base, C)], tpd[b], lsem[b])

    def wait_loads(i, b):
        base = wid * EPW + i * C
        pltpu.make_async_copy(ef_hbm.at[pl.ds(base, C)], efd[b], lsem[b]).wait()
        pltpu.make_async_copy(tp_hbm.at[pl.ds(base, C)], tpd[b], lsem[b]).wait()

    def stage(i, b):
        @pl.when(i + 1 < NCHUNK)
        def _():
            fire_loads(i + 1, 1 - b)
        wait_loads(i, b)
        pltpu.sync_copy(efd[b], accf.at[idx2.at[i]], add=True)
        pltpu.sync_copy(tpd[b], acct.at[idx2.at[i]], add=True)

    fire_loads(0, 0)

    def outer(g, carry):
        stage(2 * g, 0)
        stage(2 * g + 1, 1)
        return carry
    lax.fori_loop(0, NCHUNK // 2, outer, 0)
    stage(NCHUNK - 1, 0)
    plsc.subcore_barrier()

    def wchunk(c, carry):
        base = pl.multiple_of(jnp.minimum(sid * 640 + c * C, N - C), 8)
        pltpu.sync_copy(accf.at[pl.ds(base, C)], ef_v0)
        pltpu.sync_copy(ef_v0, outf_hbm.at[cid, pl.ds(base, C)])
        pltpu.sync_copy(acct.at[pl.ds(base, C)], tp_v0)
        pltpu.sync_copy(tp_v0, outt_hbm.at[cid, pl.ds(base, C)])
        return carry
    lax.fori_loop(0, 8, wchunk, 0)


BE = 1000
BP = 2000


def _silu(x):
    return x * (1.0 / (1.0 + jnp.exp(-x)))


def _tc_prep_body(ns_ref, wa_ref, wb_ref, p_ref, q_ref):
    ns = ns_ref[...]
    p_ref[...] = jnp.dot(ns, wa_ref[...], preferred_element_type=jnp.float32)
    q_ref[...] = jnp.dot(ns, wb_ref[...], preferred_element_type=jnp.float32)


def _tc_prep(ns, wa, wb):
    grid = (N // BP,)
    full = lambda i: (0, 0)
    return pl.pallas_call(
        _tc_prep_body,
        grid=grid,
        in_specs=[
            pl.BlockSpec((BP, D), lambda i: (i, 0)),
            pl.BlockSpec((D, H), full),
            pl.BlockSpec((D, H), full),
        ],
        out_specs=[
            pl.BlockSpec((BP, H), lambda i: (i, 0)),
            pl.BlockSpec((BP, H), lambda i: (i, 0)),
        ],
        out_shape=[
            jax.ShapeDtypeStruct((N, H), jnp.float32),
            jax.ShapeDtypeStruct((N, H), jnp.float32),
        ],
    )(ns, wa, wb)


def _tc_edge_body(hpq_ref, cd_ref, ea_ref,
                  wr_ref, we_ref, be1_ref,
                  we2_ref, be2_ref, wc1_ref, bc1_ref, wc2t_ref,
                  ef_ref, tp_ref):
    cd = cd_ref[...]
    ea = ea_ref[...]
    rad = jnp.sum(cd * cd, axis=1, keepdims=True)
    pre = (hpq_ref[...]
           + jnp.dot(ea, we_ref[...], preferred_element_type=jnp.float32)
           + rad * wr_ref[...]
           + be1_ref[...])
    h = _silu(pre)
    ef = _silu(jnp.dot(h, we2_ref[...], preferred_element_type=jnp.float32) + be2_ref[...])
    ef_ref[...] = ef
    hc = _silu(jnp.dot(ef, wc1_ref[...], preferred_element_type=jnp.float32) + bc1_ref[...])
    cm = jnp.sum(hc * wc2t_ref[...], axis=1, keepdims=True)
    trans = jnp.clip(cm * cd, -100.0, 100.0)
    ones_col = (lax.broadcasted_iota(jnp.int32, (BE, 16), 1) == 3).astype(jnp.float32)
    tp_ref[...] = trans + ones_col


def _tc_edge(hpq, cdiff, edge_attr, wr, wee, be1, we2, be2, wc1, bc1, wc2t):
    grid = (E // BE,)
    full = lambda i: (0, 0)
    return pl.pallas_call(
        _tc_edge_body,
        grid=grid,
        in_specs=[
            pl.BlockSpec((BE, D), lambda i: (i, 0)),
            pl.BlockSpec((BE, 16), lambda i: (i, 0)),
            pl.BlockSpec((BE, EA), lambda i: (i, 0)),
            pl.BlockSpec((1, H), full),
            pl.BlockSpec((EA, H), full),
            pl.BlockSpec((1, H), full),
            pl.BlockSpec((H, H), full),
            pl.BlockSpec((1, H), full),
            pl.BlockSpec((H, H), full),
            pl.BlockSpec((1, H), full),
            pl.BlockSpec((1, H), full),
        ],
        out_specs=[
            pl.BlockSpec((BE, H), lambda i: (i, 0)),
            pl.BlockSpec((BE, 16), lambda i: (i, 0)),
        ],
        out_shape=[
            jax.ShapeDtypeStruct((E, H), jnp.float32),
            jax.ShapeDtypeStruct((E, 16), jnp.float32),
        ],
    )(hpq, cdiff, edge_attr, wr, wee, be1, we2, be2, wc1, bc1, wc2t)


BN = 2000


def _tc_node_body(ns_ref, cp_ref, vp_ref, sf_ref, st_ref,
                  wn1a_ref, wn1b_ref, bn1_ref, wn2_ref, bn2_ref,
                  wv1_ref, bv1_ref, wv2t_ref, bv2_ref,
                  nsout_ref, c3_ref):
    ns = ns_ref[...]
    sf = sf_ref[0] + sf_ref[1]
    st = st_ref[0] + st_ref[1]
    cnt = jnp.maximum(st[:, 3:4], 1.0)
    agg_f = sf / cnt
    agg_t = st / cnt
    pre = (jnp.dot(ns, wn1a_ref[...], preferred_element_type=jnp.float32)
           + jnp.dot(agg_f, wn1b_ref[...], preferred_element_type=jnp.float32)
           + bn1_ref[...])
    h = _silu(pre)
    nout = jnp.dot(h, wn2_ref[...], preferred_element_type=jnp.float32) + bn2_ref[...]
    nsout_ref[...] = ns + nout
    pv = jnp.dot(ns, wv1_ref[...], preferred_element_type=jnp.float32) + bv1_ref[...]
    hv = _silu(pv)
    cvm = jnp.sum(hv * wv2t_ref[...], axis=1, keepdims=True) + bv2_ref[...]
    mask3 = (lax.broadcasted_iota(jnp.int32, (BN, 16), 1) < 3).astype(jnp.float32)
    c3_ref[...] = cp_ref[...] + agg_t * mask3 + cvm * vp_ref[...]


def _tc_node(ns, coordp, velp, sumf, sumt, wn1a, wn1b, bn1, wn2, bn2, wv1, bv1, wv2t, bv2):
    grid = (N // BN,)
    full = lambda i: (0, 0)
    return pl.pallas_call(
        _tc_node_body,
        grid=grid,
        in_specs=[
            pl.BlockSpec((BN, D), lambda i: (i, 0)),
            pl.BlockSpec((BN, 16), lambda i: (i, 0)),
            pl.BlockSpec((BN, 16), lambda i: (i, 0)),
            pl.BlockSpec((NC, BN, H), lambda i: (0, i, 0)),
            pl.BlockSpec((NC, BN, 16), lambda i: (0, i, 0)),
            pl.BlockSpec((D, H), full),
            pl.BlockSpec((H, H), full),
            pl.BlockSpec((1, H), full),
            pl.BlockSpec((H, D), full),
            pl.BlockSpec((1, D), full),
            pl.BlockSpec((D, H), full),
            pl.BlockSpec((1, H), full),
            pl.BlockSpec((1, H), full),
            pl.BlockSpec((1, 1), full),
        ],
        out_specs=[
            pl.BlockSpec((BN, D), lambda i: (i, 0)),
            pl.BlockSpec((BN, 16), lambda i: (i, 0)),
        ],
        out_shape=[
            jax.ShapeDtypeStruct((N, D), jnp.float32),
            jax.ShapeDtypeStruct((N, 16), jnp.float32),
        ],
    )(ns, coordp, velp, sumf, sumt, wn1a, wn1b, bn1, wn2, bn2, wv1, bv1, wv2t, bv2)


def kernel(node_state, edge_index, coord, velocity, edge_attr,
           We1, be1, We2, be2, Wn1, bn1, Wn2, bn2,
           Wc1, bc1, Wc2, Wv1, bv1, Wv2, bv2):
    row = edge_index[0]
    col = edge_index[1]
    row3 = row.reshape(NW, NCHUNK, C)
    col3 = col.reshape(NW, NCHUNK, C)
    coordp = jnp.pad(coord, ((0, 0), (0, 13)))
    velp = jnp.pad(velocity, ((0, 0), (0, 13)))

    wa = We1[:D]
    wb = We1[D:2 * D]
    wr = We1[2 * D:2 * D + 1]
    wee = We1[2 * D + 1:]
    p_arr, q_arr = _tc_prep(node_state, wa, wb)
    hpq, cdiff = _make_sc_gather()(p_arr, q_arr, coordp, row3, col3)

    ef, tp = _tc_edge(hpq, cdiff, edge_attr,
                      wr, wee, be1.reshape(1, H),
                      We2, be2.reshape(1, H), Wc1, bc1.reshape(1, H),
                      Wc2.reshape(1, H))

    sumf, sumt = _make_sc_scatter()(row3, ef, tp)

    ns_new, c3p = _tc_node(node_state, coordp, velp, sumf, sumt,
                           Wn1[:D], Wn1[D:], bn1.reshape(1, H),
                           Wn2, bn2.reshape(1, D),
                           Wv1, bv1.reshape(1, H), Wv2.reshape(1, H),
                           bv2.reshape(1, 1))

    c3 = c3p[:, :3].reshape(N, 3, 1)
    v3 = velocity.reshape(N, 3, 1)
    return ns_new, c3, v3


# speedup vs baseline: 1.6218x; 1.2031x over previous
import functools

import jax
import jax.numpy as jnp
from jax import lax
from jax.experimental import pallas as pl
from jax.experimental.pallas import tpu as pltpu
from jax.experimental.pallas import tpu_sc as plsc

N = 10000
E = 320000
D = 128
EA = 16
H = 128

NC, NS = 2, 16
NW = NC * NS
EPW = E // NW
C = 80
NCHUNK = EPW // C
RPT = 632

@functools.cache
def _get_mesh():
    return plsc.VectorSubcoreMesh(
        core_axis_name="c", subcore_axis_name="s", num_cores=NC, num_subcores=NS)


@functools.cache
def _make_sc_gather():
    return functools.partial(
        pl.kernel,
        out_type=(
            jax.ShapeDtypeStruct((E, D), jnp.float32),
            jax.ShapeDtypeStruct((E, 16), jnp.float32),
        ),
        mesh=_get_mesh(),
        scratch_types=[
            pltpu.VMEM((NCHUNK, C), jnp.int32),
            pltpu.VMEM((NCHUNK, C), jnp.int32),
            pltpu.VMEM((C, D), jnp.float32),
            pltpu.VMEM((C, D), jnp.float32),
            pltpu.VMEM((C, 16), jnp.float32),
            pltpu.VMEM((C, 16), jnp.float32),
            pltpu.VMEM((C, D), jnp.float32),
            pltpu.VMEM((C, D), jnp.float32),
            pltpu.VMEM((C, 16), jnp.float32),
            pltpu.VMEM((C, 16), jnp.float32),
            pltpu.SemaphoreType.DMA,
            pltpu.SemaphoreType.DMA,
            pltpu.SemaphoreType.DMA,
            pltpu.SemaphoreType.DMA,
        ],
        compiler_params=pltpu.CompilerParams(use_tc_tiling_on_sc=False),
    )(_sc_gather_body)


def _sc_gather_body(p_hbm, q_hbm, coordp_hbm, row3_hbm, col3_hbm,
                    hpq_hbm, cdiff_hbm,
                    idx_r2, idx_c2,
                    src_v0, dst_v0, cr_v0, cc_v0,
                    src_v1, dst_v1, cr_v1, cc_v1,
                    gsem0, gsem1, wsem0, wsem1):
    wid = lax.axis_index("s") * NC + lax.axis_index("c")
    data = ((src_v0, dst_v0, cr_v0, cc_v0), (src_v1, dst_v1, cr_v1, cc_v1))
    gsem = (gsem0, gsem1)
    wsem = (wsem0, wsem1)

    pltpu.sync_copy(row3_hbm.at[wid], idx_r2)
    pltpu.sync_copy(col3_hbm.at[wid], idx_c2)

    def fire(i, b):
        srcb, dstb, crb, ccb = data[b]
        pltpu.async_copy(p_hbm.at[idx_r2.at[i]], srcb, gsem[b])
        pltpu.async_copy(q_hbm.at[idx_c2.at[i]], dstb, gsem[b])
        pltpu.async_copy(coordp_hbm.at[idx_r2.at[i]], crb, gsem[b])
        pltpu.async_copy(coordp_hbm.at[idx_c2.at[i]], ccb, gsem[b])

    def wait_gathers(i, b):
        srcb, dstb, crb, ccb = data[b]
        pltpu.make_async_copy(p_hbm.at[idx_r2.at[i]], srcb, gsem[b]).wait()
        pltpu.make_async_copy(q_hbm.at[idx_c2.at[i]], dstb, gsem[b]).wait()
        pltpu.make_async_copy(coordp_hbm.at[idx_r2.at[i]], crb, gsem[b]).wait()
        pltpu.make_async_copy(coordp_hbm.at[idx_c2.at[i]], ccb, gsem[b]).wait()

    def fire_writes(i, b):
        srcb, _, crb, _ = data[b]
        base = wid * EPW + i * C
        pltpu.async_copy(srcb, hpq_hbm.at[pl.ds(base, C)], wsem[b])
        pltpu.async_copy(crb, cdiff_hbm.at[pl.ds(base, C)], wsem[b])

    def drain_writes(i, b):
        srcb, _, crb, _ = data[b]
        base = wid * EPW + i * C
        pltpu.make_async_copy(srcb, hpq_hbm.at[pl.ds(base, C)], wsem[b]).wait()
        pltpu.make_async_copy(crb, cdiff_hbm.at[pl.ds(base, C)], wsem[b]).wait()

    def compute(b):
        srcb, dstb, crb, ccb = data[b]

        @plsc.parallel_loop(0, C, unroll=4)
        def rowop(j):
            for k in range(D // 16):
                sl = pl.ds(k * 16, 16)
                plsc.addupdate(srcb.at[j, sl], dstb[j, sl])
            plsc.addupdate(crb.at[j], -ccb[j])

    def stage(i, b):
        @pl.when(i + 1 < NCHUNK)
        def _():
            @pl.when(i >= 1)
            def __():
                drain_writes(i, 1 - b)
            fire(i + 1, 1 - b)
        wait_gathers(i, b)
        compute(b)
        fire_writes(i, b)

    fire(0, 0)

    def outer(g, carry):
        stage(2 * g, 0)
        stage(2 * g + 1, 1)
        return carry
    lax.fori_loop(0, NCHUNK // 2, outer, 0)
    stage(NCHUNK - 1, 0)
    drain_writes(NCHUNK - 1, 0)
    drain_writes(NCHUNK - 2, 1)


@functools.cache
def _make_sc_scatter():
    return functools.partial(
        pl.kernel,
        out_type=(
            jax.ShapeDtypeStruct((NC, N, H), jnp.float32),
            jax.ShapeDtypeStruct((NC, N, 16), jnp.float32),
        ),
        mesh=_get_mesh(),
        scratch_types=[
            pltpu.VMEM((NCHUNK, C), jnp.int32),
            pltpu.VMEM((C, H), jnp.float32),
            pltpu.VMEM((C, 16), jnp.float32),
            pltpu.VMEM((C, H), jnp.float32),
            pltpu.VMEM((C, 16), jnp.float32),
            pltpu.VMEM_SHARED((N, H), jnp.float32),
            pltpu.VMEM_SHARED((N, 16), jnp.float32),
            pltpu.SemaphoreType.DMA,
            pltpu.SemaphoreType.DMA,
        ],
        compiler_params=pltpu.CompilerParams(use_tc_tiling_on_sc=False),
    )(_sc_scatter_body)


def _sc_scatter_body(row3_hbm, ef_hbm, tp_hbm,
                     outf_hbm, outt_hbm,
                     idx2, ef_v0, tp_v0, ef_v1, tp_v1, accf, acct,
                     lsem0, lsem1):
    cid = lax.axis_index("c")
    sid = lax.axis_index("s")
    wid = sid * NC + cid
    efd = (ef_v0, ef_v1)
    tpd = (tp_v0, tp_v1)
    lsem = (lsem0, lsem1)

    pltpu.sync_copy(row3_hbm.at[wid], idx2)

    def zrow(j, carry):
        def zcol(k, c2):
            ef_v0[j, pl.ds(k * 16, 16)] = jnp.zeros((16,), jnp.float32)
            return c2
        lax.fori_loop(0, H // 16, zcol, 0)
        tp_v0[j] = jnp.zeros((16,), jnp.float32)
        return carry
    lax.fori_loop(0, C, zrow, 0)

    def zchunk(c, carry):
        base = pl.multiple_of(jnp.minimum(sid * 640 + c * C, N - C), 8)
        pltpu.sync_copy(ef_v0, accf.at[pl.ds(base, C)])
        pltpu.sync_copy(tp_v0, acct.at[pl.ds(base, C)])
        return carry
    lax.fori_loop(0, 8, zchunk, 0)
    plsc.subcore_barrier()

    def fire_loads(i, b):
        base = wid * EPW + i * C
        pltpu.async_copy(ef_hbm.at[pl.ds(base, C)], efd[b], lsem[b])
        pltpu.async_copy(tp_hbm.at[pl.ds(base, C)], tpd[b], lsem[b])

    def wait_loads(i, b):
        base = wid * EPW + i * C
        pltpu.make_async_copy(ef_hbm.at[pl.ds(base, C)], efd[b], lsem[b]).wait()
        pltpu.make_async_copy(tp_hbm.at[pl.ds(base, C)], tpd[b], lsem[b]).wait()

    def stage(i, b):
        @pl.when(i + 1 < NCHUNK)
        def _():
            fire_loads(i + 1, 1 - b)
        wait_loads(i, b)
        pltpu.sync_copy(efd[b], accf.at[idx2.at[i]], add=True)
        pltpu.sync_copy(tpd[b], acct.at[idx2.at[i]], add=True)

    fire_loads(0, 0)

    def outer(g, carry):
        stage(2 * g, 0)
        stage(2 * g + 1, 1)
        return carry
    lax.fori_loop(0, NCHUNK // 2, outer, 0)
    stage(NCHUNK - 1, 0)
    plsc.subcore_barrier()

    def wchunk(c, carry):
        base = pl.multiple_of(jnp.minimum(sid * 640 + c * C, N - C), 8)
        pltpu.sync_copy(accf.at[pl.ds(base, C)], ef_v0)
        pltpu.sync_copy(ef_v0, outf_hbm.at[cid, pl.ds(base, C)])
        pltpu.sync_copy(acct.at[pl.ds(base, C)], tp_v0)
        pltpu.sync_copy(tp_v0, outt_hbm.at[cid, pl.ds(base, C)])
        return carry
    lax.fori_loop(0, 8, wchunk, 0)


BE = 1000
BP = 2000


def _silu(x):
    return x * (1.0 / (1.0 + jnp.exp(-x)))


def _tc_prep_body(ns_ref, wa_ref, wb_ref, p_ref, q_ref):
    ns = ns_ref[...]
    p_ref[...] = jnp.dot(ns, wa_ref[...], preferred_element_type=jnp.float32)
    q_ref[...] = jnp.dot(ns, wb_ref[...], preferred_element_type=jnp.float32)


def _tc_prep(ns, wa, wb):
    grid = (N // BP,)
    full = lambda i: (0, 0)
    return pl.pallas_call(
        _tc_prep_body,
        grid=grid,
        in_specs=[
            pl.BlockSpec((BP, D), lambda i: (i, 0)),
            pl.BlockSpec((D, H), full),
            pl.BlockSpec((D, H), full),
        ],
        out_specs=[
            pl.BlockSpec((BP, H), lambda i: (i, 0)),
            pl.BlockSpec((BP, H), lambda i: (i, 0)),
        ],
        out_shape=[
            jax.ShapeDtypeStruct((N, H), jnp.float32),
            jax.ShapeDtypeStruct((N, H), jnp.float32),
        ],
    )(ns, wa, wb)


def _tc_edge_body(hpq_ref, cd_ref, ea_ref,
                  wr_ref, we_ref, be1_ref,
                  we2_ref, be2_ref, wc1_ref, bc1_ref, wc2t_ref,
                  ef_ref, tp_ref):
    cd = cd_ref[...]
    ea = ea_ref[...]
    rad = jnp.sum(cd * cd, axis=1, keepdims=True)
    pre = (hpq_ref[...]
           + jnp.dot(ea, we_ref[...], preferred_element_type=jnp.float32)
           + rad * wr_ref[...]
           + be1_ref[...])
    h = _silu(pre)
    ef = _silu(jnp.dot(h, we2_ref[...], preferred_element_type=jnp.float32) + be2_ref[...])
    ef_ref[...] = ef
    hc = _silu(jnp.dot(ef, wc1_ref[...], preferred_element_type=jnp.float32) + bc1_ref[...])
    cm = jnp.sum(hc * wc2t_ref[...], axis=1, keepdims=True)
    trans = jnp.clip(cm * cd, -100.0, 100.0)
    ones_col = (lax.broadcasted_iota(jnp.int32, (BE, 16), 1) == 3).astype(jnp.float32)
    tp_ref[...] = trans + ones_col


def _tc_edge(hpq, cdiff, edge_attr, wr, wee, be1, we2, be2, wc1, bc1, wc2t):
    grid = (E // BE,)
    full = lambda i: (0, 0)
    return pl.pallas_call(
        _tc_edge_body,
        grid=grid,
        in_specs=[
            pl.BlockSpec((BE, D), lambda i: (i, 0)),
            pl.BlockSpec((BE, 16), lambda i: (i, 0)),
            pl.BlockSpec((BE, EA), lambda i: (i, 0)),
            pl.BlockSpec((1, H), full),
            pl.BlockSpec((EA, H), full),
            pl.BlockSpec((1, H), full),
            pl.BlockSpec((H, H), full),
            pl.BlockSpec((1, H), full),
            pl.BlockSpec((H, H), full),
            pl.BlockSpec((1, H), full),
            pl.BlockSpec((1, H), full),
        ],
        out_specs=[
            pl.BlockSpec((BE, H), lambda i: (i, 0)),
            pl.BlockSpec((BE, 16), lambda i: (i, 0)),
        ],
        out_shape=[
            jax.ShapeDtypeStruct((E, H), jnp.float32),
            jax.ShapeDtypeStruct((E, 16), jnp.float32),
        ],
    )(hpq, cdiff, edge_attr, wr, wee, be1, we2, be2, wc1, bc1, wc2t)


BN = 2000


def _tc_node_body(ns_ref, cp_ref, vp_ref, sf_ref, st_ref,
                  wn1a_ref, wn1b_ref, bn1_ref, wn2_ref, bn2_ref,
                  wv1_ref, bv1_ref, wv2t_ref, bv2_ref,
                  nsout_ref, c3_ref):
    ns = ns_ref[...]
    sf = sf_ref[0] + sf_ref[1]
    st = st_ref[0] + st_ref[1]
    cnt = jnp.maximum(st[:, 3:4], 1.0)
    agg_f = sf / cnt
    agg_t = st / cnt
    pre = (jnp.dot(ns, wn1a_ref[...], preferred_element_type=jnp.float32)
           + jnp.dot(agg_f, wn1b_ref[...], preferred_element_type=jnp.float32)
           + bn1_ref[...])
    h = _silu(pre)
    nout = jnp.dot(h, wn2_ref[...], preferred_element_type=jnp.float32) + bn2_ref[...]
    nsout_ref[...] = ns + nout
    pv = jnp.dot(ns, wv1_ref[...], preferred_element_type=jnp.float32) + bv1_ref[...]
    hv = _silu(pv)
    cvm = jnp.sum(hv * wv2t_ref[...], axis=1, keepdims=True) + bv2_ref[...]
    mask3 = (lax.broadcasted_iota(jnp.int32, (BN, 16), 1) < 3).astype(jnp.float32)
    c3_ref[...] = cp_ref[...] + agg_t * mask3 + cvm * vp_ref[...]


def _tc_node(ns, coordp, velp, sumf, sumt, wn1a, wn1b, bn1, wn2, bn2, wv1, bv1, wv2t, bv2):
    grid = (N // BN,)
    full = lambda i: (0, 0)
    return pl.pallas_call(
        _tc_node_body,
        grid=grid,
        in_specs=[
            pl.BlockSpec((BN, D), lambda i: (i, 0)),
            pl.BlockSpec((BN, 16), lambda i: (i, 0)),
            pl.BlockSpec((BN, 16), lambda i: (i, 0)),
            pl.BlockSpec((NC, BN, H), lambda i: (0, i, 0)),
            pl.BlockSpec((NC, BN, 16), lambda i: (0, i, 0)),
            pl.BlockSpec((D, H), full),
            pl.BlockSpec((H, H), full),
            pl.BlockSpec((1, H), full),
            pl.BlockSpec((H, D), full),
            pl.BlockSpec((1, D), full),
            pl.BlockSpec((D, H), full),
            pl.BlockSpec((1, H), full),
            pl.BlockSpec((1, H), full),
            pl.BlockSpec((1, 1), full),
        ],
        out_specs=[
            pl.BlockSpec((BN, D), lambda i: (i, 0)),
            pl.BlockSpec((BN, 16), lambda i: (i, 0)),
        ],
        out_shape=[
            jax.ShapeDtypeStruct((N, D), jnp.float32),
            jax.ShapeDtypeStruct((N, 16), jnp.float32),
        ],
    )(ns, coordp, velp, sumf, sumt, wn1a, wn1b, bn1, wn2, bn2, wv1, bv1, wv2t, bv2)


def kernel(node_state, edge_index, coord, velocity, edge_attr,
           We1, be1, We2, be2, Wn1, bn1, Wn2, bn2,
           Wc1, bc1, Wc2, Wv1, bv1, Wv2, bv2):
    row = edge_index[0]
    col = edge_index[1]
    row3 = row.reshape(NW, NCHUNK, C)
    col3 = col.reshape(NW, NCHUNK, C)
    coordp = jnp.pad(coord, ((0, 0), (0, 13)))
    velp = jnp.pad(velocity, ((0, 0), (0, 13)))

    wa = We1[:D]
    wb = We1[D:2 * D]
    wr = We1[2 * D:2 * D + 1]
    wee = We1[2 * D + 1:]
    p_arr, q_arr = _tc_prep(node_state, wa, wb)
    hpq, cdiff = _make_sc_gather()(p_arr, q_arr, coordp, row3, col3)

    ef, tp = _tc_edge(hpq, cdiff, edge_attr,
                      wr, wee, be1.reshape(1, H),
                      We2, be2.reshape(1, H), Wc1, bc1.reshape(1, H),
                      Wc2.reshape(1, H))

    sumf, sumt = _make_sc_scatter()(row3, ef, tp)

    ns_new, c3p = _tc_node(node_state, coordp, velp, sumf, sumt,
                           Wn1[:D], Wn1[D:], bn1.reshape(1, H),
                           Wn2, bn2.reshape(1, D),
                           Wv1, bv1.reshape(1, H), Wv2.reshape(1, H),
                           bv2.reshape(1, 1))

    c3 = c3p[:, :3].reshape(N, 3, 1)
    v3 = velocity.reshape(N, 3, 1)
    return ns_new, c3, v3


# speedup vs baseline: 1.8280x; 1.1272x over previous
import functools

import jax
import jax.numpy as jnp
from jax import lax
from jax.experimental import pallas as pl
from jax.experimental.pallas import tpu as pltpu
from jax.experimental.pallas import tpu_sc as plsc

N = 10000
E = 320000
D = 128
EA = 16
H = 128

NC, NS = 2, 16
NW = NC * NS
EPW = E // NW
C = 80
NCHUNK = EPW // C
RPT = 632

@functools.cache
def _get_mesh():
    return plsc.VectorSubcoreMesh(
        core_axis_name="c", subcore_axis_name="s", num_cores=NC, num_subcores=NS)


@functools.cache
def _make_sc_gather():
    return functools.partial(
        pl.kernel,
        out_type=(
            jax.ShapeDtypeStruct((E, D), jnp.float32),
            jax.ShapeDtypeStruct((E, 16), jnp.float32),
        ),
        mesh=_get_mesh(),
        scratch_types=[
            pltpu.VMEM((NCHUNK, C), jnp.int32),
            pltpu.VMEM((NCHUNK, C), jnp.int32),
            pltpu.VMEM((C, D), jnp.float32),
            pltpu.VMEM((C, D), jnp.float32),
            pltpu.VMEM((C, 16), jnp.float32),
            pltpu.VMEM((C, 16), jnp.float32),
            pltpu.VMEM((C, D), jnp.float32),
            pltpu.VMEM((C, D), jnp.float32),
            pltpu.VMEM((C, 16), jnp.float32),
            pltpu.VMEM((C, 16), jnp.float32),
            pltpu.SemaphoreType.DMA,
            pltpu.SemaphoreType.DMA,
            pltpu.SemaphoreType.DMA,
            pltpu.SemaphoreType.DMA,
        ],
        compiler_params=pltpu.CompilerParams(use_tc_tiling_on_sc=False),
    )(_sc_gather_body)


def _sc_gather_body(p_hbm, q_hbm, coordp_hbm, row3_hbm, col3_hbm,
                    hpq_hbm, cdiff_hbm,
                    idx_r2, idx_c2,
                    src_v0, dst_v0, cr_v0, cc_v0,
                    src_v1, dst_v1, cr_v1, cc_v1,
                    gsem0, gsem1, wsem0, wsem1):
    wid = lax.axis_index("s") * NC + lax.axis_index("c")
    data = ((src_v0, dst_v0, cr_v0, cc_v0), (src_v1, dst_v1, cr_v1, cc_v1))
    gsem = (gsem0, gsem1)
    wsem = (wsem0, wsem1)

    pltpu.sync_copy(row3_hbm.at[wid], idx_r2)
    pltpu.sync_copy(col3_hbm.at[wid], idx_c2)

    def fire(i, b):
        srcb, dstb, crb, ccb = data[b]
        pltpu.async_copy(p_hbm.at[idx_r2.at[i]], srcb, gsem[b])
        pltpu.async_copy(q_hbm.at[idx_c2.at[i]], dstb, gsem[b])
        pltpu.async_copy(coordp_hbm.at[idx_r2.at[i]], crb, gsem[b])
        pltpu.async_copy(coordp_hbm.at[idx_c2.at[i]], ccb, gsem[b])

    def wait_gathers(i, b):
        srcb, dstb, crb, ccb = data[b]
        pltpu.make_async_copy(p_hbm.at[idx_r2.at[i]], srcb, gsem[b]).wait()
        pltpu.make_async_copy(q_hbm.at[idx_c2.at[i]], dstb, gsem[b]).wait()
        pltpu.make_async_copy(coordp_hbm.at[idx_r2.at[i]], crb, gsem[b]).wait()
        pltpu.make_async_copy(coordp_hbm.at[idx_c2.at[i]], ccb, gsem[b]).wait()

    def fire_writes(i, b):
        srcb, _, crb, _ = data[b]
        base = wid * EPW + i * C
        pltpu.async_copy(srcb, hpq_hbm.at[pl.ds(base, C)], wsem[b])
        pltpu.async_copy(crb, cdiff_hbm.at[pl.ds(base, C)], wsem[b])

    def drain_writes(i, b):
        srcb, _, crb, _ = data[b]
        base = wid * EPW + i * C
        pltpu.make_async_copy(srcb, hpq_hbm.at[pl.ds(base, C)], wsem[b]).wait()
        pltpu.make_async_copy(crb, cdiff_hbm.at[pl.ds(base, C)], wsem[b]).wait()

    def compute(b):
        srcb, dstb, crb, ccb = data[b]

        @plsc.parallel_loop(0, C, unroll=4)
        def rowop(j):
            for k in range(D // 16):
                sl = pl.ds(k * 16, 16)
                plsc.addupdate(srcb.at[j, sl], dstb[j, sl])
            plsc.addupdate(crb.at[j], -ccb[j])

    def stage(i, b):
        @pl.when(i + 1 < NCHUNK)
        def _():
            @pl.when(i >= 1)
            def __():
                drain_writes(i, 1 - b)
            fire(i + 1, 1 - b)
        wait_gathers(i, b)
        compute(b)
        fire_writes(i, b)

    fire(0, 0)

    def outer(g, carry):
        stage(2 * g, 0)
        stage(2 * g + 1, 1)
        return carry
    lax.fori_loop(0, NCHUNK // 2, outer, 0)
    stage(NCHUNK - 1, 0)
    drain_writes(NCHUNK - 1, 0)
    drain_writes(NCHUNK - 2, 1)


@functools.cache
def _make_sc_scatter():
    return functools.partial(
        pl.kernel,
        out_type=(
            jax.ShapeDtypeStruct((NC, N, H), jnp.float32),
            jax.ShapeDtypeStruct((NC, N, 16), jnp.float32),
        ),
        mesh=_get_mesh(),
        scratch_types=[
            pltpu.VMEM((NCHUNK, C), jnp.int32),
            pltpu.VMEM((C, H), jnp.float32),
            pltpu.VMEM((C, 16), jnp.float32),
            pltpu.VMEM((C, H), jnp.float32),
            pltpu.VMEM((C, 16), jnp.float32),
            pltpu.VMEM_SHARED((N, H), jnp.float32),
            pltpu.VMEM_SHARED((N, 16), jnp.float32),
            pltpu.SemaphoreType.DMA,
            pltpu.SemaphoreType.DMA,
        ],
        compiler_params=pltpu.CompilerParams(use_tc_tiling_on_sc=False),
    )(_sc_scatter_body)


def _sc_scatter_body(row3_hbm, ef_hbm, tp_hbm,
                     outf_hbm, outt_hbm,
                     idx2, ef_v0, tp_v0, ef_v1, tp_v1, accf, acct,
                     lsem0, lsem1):
    cid = lax.axis_index("c")
    sid = lax.axis_index("s")
    wid = sid * NC + cid
    efd = (ef_v0, ef_v1)
    tpd = (tp_v0, tp_v1)
    lsem = (lsem0, lsem1)

    pltpu.sync_copy(row3_hbm.at[wid], idx2)

    def zrow(j, carry):
        def zcol(k, c2):
            ef_v0[j, pl.ds(k * 16, 16)] = jnp.zeros((16,), jnp.float32)
            return c2
        lax.fori_loop(0, H // 16, zcol, 0)
        tp_v0[j] = jnp.zeros((16,), jnp.float32)
        return carry
    lax.fori_loop(0, C, zrow, 0)

    def zchunk(c, carry):
        base = pl.multiple_of(jnp.minimum(sid * 640 + c * C, N - C), 8)
        pltpu.sync_copy(ef_v0, accf.at[pl.ds(base, C)])
        pltpu.sync_copy(tp_v0, acct.at[pl.ds(base, C)])
        return carry
    lax.fori_loop(0, 8, zchunk, 0)
    plsc.subcore_barrier()

    def fire_loads(i, b):
        base = wid * EPW + i * C
        pltpu.async_copy(ef_hbm.at[pl.ds(base, C)], efd[b], lsem[b])
        pltpu.async_copy(tp_hbm.at[pl.ds(base, C)], tpd[b], lsem[b])

    def wait_loads(i, b):
        base = wid * EPW + i * C
        pltpu.make_async_copy(ef_hbm.at[pl.ds(base, C)], efd[b], lsem[b]).wait()
        pltpu.make_async_copy(tp_hbm.at[pl.ds(base, C)], tpd[b], lsem[b]).wait()

    def stage(i, b):
        @pl.when(i + 1 < NCHUNK)
        def _():
            fire_loads(i + 1, 1 - b)
        wait_loads(i, b)
        pltpu.sync_copy(efd[b], accf.at[idx2.at[i]], add=True)
        pltpu.sync_copy(tpd[b], acct.at[idx2.at[i]], add=True)

    fire_loads(0, 0)

    def outer(g, carry):
        stage(2 * g, 0)
        stage(2 * g + 1, 1)
        return carry
    lax.fori_loop(0, NCHUNK // 2, outer, 0)
    stage(NCHUNK - 1, 0)
    plsc.subcore_barrier()

    def wchunk(c, carry):
        base = pl.multiple_of(jnp.minimum(sid * 640 + c * C, N - C), 8)
        pltpu.sync_copy(accf.at[pl.ds(base, C)], ef_v0)
        pltpu.sync_copy(ef_v0, outf_hbm.at[cid, pl.ds(base, C)])
        pltpu.sync_copy(acct.at[pl.ds(base, C)], tp_v0)
        pltpu.sync_copy(tp_v0, outt_hbm.at[cid, pl.ds(base, C)])
        return carry
    lax.fori_loop(0, 8, wchunk, 0)


BE = 2000
BP = 2000


def _silu(x):
    return x * (1.0 / (1.0 + jnp.exp(-x)))


def _tc_prep_body(ns_ref, wa_ref, wb_ref, p_ref, q_ref):
    ns = ns_ref[...]
    p_ref[...] = jnp.dot(ns, wa_ref[...], preferred_element_type=jnp.float32)
    q_ref[...] = jnp.dot(ns, wb_ref[...], preferred_element_type=jnp.float32)


def _tc_prep(ns, wa, wb):
    grid = (N // BP,)
    full = lambda i: (0, 0)
    return pl.pallas_call(
        _tc_prep_body,
        grid=grid,
        in_specs=[
            pl.BlockSpec((BP, D), lambda i: (i, 0)),
            pl.BlockSpec((D, H), full),
            pl.BlockSpec((D, H), full),
        ],
        out_specs=[
            pl.BlockSpec((BP, H), lambda i: (i, 0)),
            pl.BlockSpec((BP, H), lambda i: (i, 0)),
        ],
        out_shape=[
            jax.ShapeDtypeStruct((N, H), jnp.float32),
            jax.ShapeDtypeStruct((N, H), jnp.float32),
        ],
    )(ns, wa, wb)


def _tc_edge_body(hpq_ref, cd_ref, ea_ref,
                  wr_ref, we_ref, be1_ref,
                  we2_ref, be2_ref, wc1_ref, bc1_ref, wc2t_ref,
                  ef_ref, tp_ref):
    cd = cd_ref[...]
    ea = ea_ref[...]
    rad = jnp.sum(cd * cd, axis=1, keepdims=True)
    pre = (hpq_ref[...]
           + jnp.dot(ea, we_ref[...], preferred_element_type=jnp.float32)
           + rad * wr_ref[...]
           + be1_ref[...])
    h = _silu(pre)
    ef = _silu(jnp.dot(h, we2_ref[...], preferred_element_type=jnp.float32) + be2_ref[...])
    ef_ref[...] = ef
    hc = _silu(jnp.dot(ef, wc1_ref[...], preferred_element_type=jnp.float32) + bc1_ref[...])
    cm = jnp.sum(hc * wc2t_ref[...], axis=1, keepdims=True)
    trans = jnp.clip(cm * cd, -100.0, 100.0)
    ones_col = (lax.broadcasted_iota(jnp.int32, (BE, 16), 1) == 3).astype(jnp.float32)
    tp_ref[...] = trans + ones_col


def _tc_edge(hpq, cdiff, edge_attr, wr, wee, be1, we2, be2, wc1, bc1, wc2t):
    grid = (E // BE,)
    full = lambda i: (0, 0)
    return pl.pallas_call(
        _tc_edge_body,
        grid=grid,
        in_specs=[
            pl.BlockSpec((BE, D), lambda i: (i, 0)),
            pl.BlockSpec((BE, 16), lambda i: (i, 0)),
            pl.BlockSpec((BE, EA), lambda i: (i, 0)),
            pl.BlockSpec((1, H), full),
            pl.BlockSpec((EA, H), full),
            pl.BlockSpec((1, H), full),
            pl.BlockSpec((H, H), full),
            pl.BlockSpec((1, H), full),
            pl.BlockSpec((H, H), full),
            pl.BlockSpec((1, H), full),
            pl.BlockSpec((1, H), full),
        ],
        out_specs=[
            pl.BlockSpec((BE, H), lambda i: (i, 0)),
            pl.BlockSpec((BE, 16), lambda i: (i, 0)),
        ],
        out_shape=[
            jax.ShapeDtypeStruct((E, H), jnp.float32),
            jax.ShapeDtypeStruct((E, 16), jnp.float32),
        ],
    )(hpq, cdiff, edge_attr, wr, wee, be1, we2, be2, wc1, bc1, wc2t)


BN = 2000


def _tc_node_body(ns_ref, cp_ref, vp_ref, sf_ref, st_ref,
                  wn1a_ref, wn1b_ref, bn1_ref, wn2_ref, bn2_ref,
                  wv1_ref, bv1_ref, wv2t_ref, bv2_ref,
                  nsout_ref, c3_ref):
    ns = ns_ref[...]
    sf = sf_ref[0] + sf_ref[1]
    st = st_ref[0] + st_ref[1]
    cnt = jnp.maximum(st[:, 3:4], 1.0)
    agg_f = sf / cnt
    agg_t = st / cnt
    pre = (jnp.dot(ns, wn1a_ref[...], preferred_element_type=jnp.float32)
           + jnp.dot(agg_f, wn1b_ref[...], preferred_element_type=jnp.float32)
           + bn1_ref[...])
    h = _silu(pre)
    nout = jnp.dot(h, wn2_ref[...], preferred_element_type=jnp.float32) + bn2_ref[...]
    nsout_ref[...] = ns + nout
    pv = jnp.dot(ns, wv1_ref[...], preferred_element_type=jnp.float32) + bv1_ref[...]
    hv = _silu(pv)
    cvm = jnp.sum(hv * wv2t_ref[...], axis=1, keepdims=True) + bv2_ref[...]
    mask3 = (lax.broadcasted_iota(jnp.int32, (BN, 16), 1) < 3).astype(jnp.float32)
    c3_ref[...] = cp_ref[...] + agg_t * mask3 + cvm * vp_ref[...]


def _tc_node(ns, coordp, velp, sumf, sumt, wn1a, wn1b, bn1, wn2, bn2, wv1, bv1, wv2t, bv2):
    grid = (N // BN,)
    full = lambda i: (0, 0)
    return pl.pallas_call(
        _tc_node_body,
        grid=grid,
        in_specs=[
            pl.BlockSpec((BN, D), lambda i: (i, 0)),
            pl.BlockSpec((BN, 16), lambda i: (i, 0)),
            pl.BlockSpec((BN, 16), lambda i: (i, 0)),
            pl.BlockSpec((NC, BN, H), lambda i: (0, i, 0)),
            pl.BlockSpec((NC, BN, 16), lambda i: (0, i, 0)),
            pl.BlockSpec((D, H), full),
            pl.BlockSpec((H, H), full),
            pl.BlockSpec((1, H), full),
            pl.BlockSpec((H, D), full),
            pl.BlockSpec((1, D), full),
            pl.BlockSpec((D, H), full),
            pl.BlockSpec((1, H), full),
            pl.BlockSpec((1, H), full),
            pl.BlockSpec((1, 1), full),
        ],
        out_specs=[
            pl.BlockSpec((BN, D), lambda i: (i, 0)),
            pl.BlockSpec((BN, 16), lambda i: (i, 0)),
        ],
        out_shape=[
            jax.ShapeDtypeStruct((N, D), jnp.float32),
            jax.ShapeDtypeStruct((N, 16), jnp.float32),
        ],
    )(ns, coordp, velp, sumf, sumt, wn1a, wn1b, bn1, wn2, bn2, wv1, bv1, wv2t, bv2)


def kernel(node_state, edge_index, coord, velocity, edge_attr,
           We1, be1, We2, be2, Wn1, bn1, Wn2, bn2,
           Wc1, bc1, Wc2, Wv1, bv1, Wv2, bv2):
    row = edge_index[0]
    col = edge_index[1]
    row3 = row.reshape(NW, NCHUNK, C)
    col3 = col.reshape(NW, NCHUNK, C)
    coordp = jnp.pad(coord, ((0, 0), (0, 13)))
    velp = jnp.pad(velocity, ((0, 0), (0, 13)))

    wa = We1[:D]
    wb = We1[D:2 * D]
    wr = We1[2 * D:2 * D + 1]
    wee = We1[2 * D + 1:]
    p_arr, q_arr = _tc_prep(node_state, wa, wb)
    hpq, cdiff = _make_sc_gather()(p_arr, q_arr, coordp, row3, col3)

    ef, tp = _tc_edge(hpq, cdiff, edge_attr,
                      wr, wee, be1.reshape(1, H),
                      We2, be2.reshape(1, H), Wc1, bc1.reshape(1, H),
                      Wc2.reshape(1, H))

    sumf, sumt = _make_sc_scatter()(row3, ef, tp)

    ns_new, c3p = _tc_node(node_state, coordp, velp, sumf, sumt,
                           Wn1[:D], Wn1[D:], bn1.reshape(1, H),
                           Wn2, bn2.reshape(1, D),
                           Wv1, bv1.reshape(1, H), Wv2.reshape(1, H),
                           bv2.reshape(1, 1))

    c3 = c3p[:, :3].reshape(N, 3, 1)
    v3 = velocity.reshape(N, 3, 1)
    return ns_new, c3, v3


# speedup vs baseline: 1.9127x; 1.0463x over previous
import functools

import jax
import jax.numpy as jnp
from jax import lax
from jax.experimental import pallas as pl
from jax.experimental.pallas import tpu as pltpu
from jax.experimental.pallas import tpu_sc as plsc

N = 10000
E = 320000
D = 128
EA = 16
H = 128

NC, NS = 2, 16
NW = NC * NS
EPW = E // NW
C = 80
NCHUNK = EPW // C
RPT = 632

@functools.cache
def _get_mesh():
    return plsc.VectorSubcoreMesh(
        core_axis_name="c", subcore_axis_name="s", num_cores=NC, num_subcores=NS)


@functools.cache
def _make_sc_gather():
    return functools.partial(
        pl.kernel,
        out_type=(
            jax.ShapeDtypeStruct((E, D), jnp.float32),
            jax.ShapeDtypeStruct((E, 16), jnp.float32),
        ),
        mesh=_get_mesh(),
        scratch_types=[
            pltpu.VMEM((NCHUNK, C), jnp.int32),
            pltpu.VMEM((NCHUNK, C), jnp.int32),
            pltpu.VMEM((C, D), jnp.float32),
            pltpu.VMEM((C, D), jnp.float32),
            pltpu.VMEM((C, 16), jnp.float32),
            pltpu.VMEM((C, 16), jnp.float32),
            pltpu.VMEM((C, D), jnp.float32),
            pltpu.VMEM((C, D), jnp.float32),
            pltpu.VMEM((C, 16), jnp.float32),
            pltpu.VMEM((C, 16), jnp.float32),
            pltpu.SemaphoreType.DMA,
            pltpu.SemaphoreType.DMA,
            pltpu.SemaphoreType.DMA,
            pltpu.SemaphoreType.DMA,
        ],
        compiler_params=pltpu.CompilerParams(use_tc_tiling_on_sc=False),
    )(_sc_gather_body)


def _sc_gather_body(p_hbm, q_hbm, coordp_hbm, row3_hbm, col3_hbm,
                    hpq_hbm, cdiff_hbm,
                    idx_r2, idx_c2,
                    src_v0, dst_v0, cr_v0, cc_v0,
                    src_v1, dst_v1, cr_v1, cc_v1,
                    gsem0, gsem1, wsem0, wsem1):
    wid = lax.axis_index("s") * NC + lax.axis_index("c")
    data = ((src_v0, dst_v0, cr_v0, cc_v0), (src_v1, dst_v1, cr_v1, cc_v1))
    gsem = (gsem0, gsem1)
    wsem = (wsem0, wsem1)

    pltpu.sync_copy(row3_hbm.at[wid], idx_r2)
    pltpu.sync_copy(col3_hbm.at[wid], idx_c2)

    def fire(i, b):
        srcb, dstb, crb, ccb = data[b]
        pltpu.async_copy(p_hbm.at[idx_r2.at[i]], srcb, gsem[b])
        pltpu.async_copy(q_hbm.at[idx_c2.at[i]], dstb, gsem[b])
        pltpu.async_copy(coordp_hbm.at[idx_r2.at[i]], crb, gsem[b])
        pltpu.async_copy(coordp_hbm.at[idx_c2.at[i]], ccb, gsem[b])

    def wait_gathers(i, b):
        srcb, dstb, crb, ccb = data[b]
        pltpu.make_async_copy(p_hbm.at[idx_r2.at[i]], srcb, gsem[b]).wait()
        pltpu.make_async_copy(q_hbm.at[idx_c2.at[i]], dstb, gsem[b]).wait()
        pltpu.make_async_copy(coordp_hbm.at[idx_r2.at[i]], crb, gsem[b]).wait()
        pltpu.make_async_copy(coordp_hbm.at[idx_c2.at[i]], ccb, gsem[b]).wait()

    def fire_writes(i, b):
        srcb, _, crb, _ = data[b]
        base = wid * EPW + i * C
        pltpu.async_copy(srcb, hpq_hbm.at[pl.ds(base, C)], wsem[b])
        pltpu.async_copy(crb, cdiff_hbm.at[pl.ds(base, C)], wsem[b])

    def drain_writes(i, b):
        srcb, _, crb, _ = data[b]
        base = wid * EPW + i * C
        pltpu.make_async_copy(srcb, hpq_hbm.at[pl.ds(base, C)], wsem[b]).wait()
        pltpu.make_async_copy(crb, cdiff_hbm.at[pl.ds(base, C)], wsem[b]).wait()

    def compute(b):
        srcb, dstb, crb, ccb = data[b]

        @plsc.parallel_loop(0, C, unroll=4)
        def rowop(j):
            for k in range(D // 16):
                sl = pl.ds(k * 16, 16)
                plsc.addupdate(srcb.at[j, sl], dstb[j, sl])
            plsc.addupdate(crb.at[j], -ccb[j])

    def stage(i, b):
        @pl.when(i + 1 < NCHUNK)
        def _():
            @pl.when(i >= 1)
            def __():
                drain_writes(i, 1 - b)
            fire(i + 1, 1 - b)
        wait_gathers(i, b)
        compute(b)
        fire_writes(i, b)

    fire(0, 0)

    def outer(g, carry):
        stage(2 * g, 0)
        stage(2 * g + 1, 1)
        return carry
    lax.fori_loop(0, NCHUNK // 2, outer, 0)
    stage(NCHUNK - 1, 0)
    drain_writes(NCHUNK - 1, 0)
    drain_writes(NCHUNK - 2, 1)


@functools.cache
def _make_sc_scatter():
    return functools.partial(
        pl.kernel,
        out_type=(
            jax.ShapeDtypeStruct((NC, N, H), jnp.float32),
            jax.ShapeDtypeStruct((NC, N, 16), jnp.float32),
        ),
        mesh=_get_mesh(),
        scratch_types=[
            pltpu.VMEM((NCHUNK, C), jnp.int32),
            pltpu.VMEM((C, H), jnp.float32),
            pltpu.VMEM((C, 16), jnp.float32),
            pltpu.VMEM((C, H), jnp.float32),
            pltpu.VMEM((C, 16), jnp.float32),
            pltpu.VMEM_SHARED((N, H), jnp.float32),
            pltpu.VMEM_SHARED((N, 16), jnp.float32),
            pltpu.SemaphoreType.DMA,
            pltpu.SemaphoreType.DMA,
        ],
        compiler_params=pltpu.CompilerParams(use_tc_tiling_on_sc=False),
    )(_sc_scatter_body)


def _sc_scatter_body(row3_hbm, ef_hbm, tp_hbm,
                     outf_hbm, outt_hbm,
                     idx2, ef_v0, tp_v0, ef_v1, tp_v1, accf, acct,
                     lsem0, lsem1):
    cid = lax.axis_index("c")
    sid = lax.axis_index("s")
    wid = sid * NC + cid
    efd = (ef_v0, ef_v1)
    tpd = (tp_v0, tp_v1)
    lsem = (lsem0, lsem1)

    pltpu.sync_copy(row3_hbm.at[wid], idx2)

    def zrow(j, carry):
        def zcol(k, c2):
            ef_v0[j, pl.ds(k * 16, 16)] = jnp.zeros((16,), jnp.float32)
            return c2
        lax.fori_loop(0, H // 16, zcol, 0)
        tp_v0[j] = jnp.zeros((16,), jnp.float32)
        return carry
    lax.fori_loop(0, C, zrow, 0)

    def zchunk(c, carry):
        base = pl.multiple_of(jnp.minimum(sid * 640 + c * C, N - C), 8)
        pltpu.sync_copy(ef_v0, accf.at[pl.ds(base, C)])
        pltpu.sync_copy(tp_v0, acct.at[pl.ds(base, C)])
        return carry
    lax.fori_loop(0, 8, zchunk, 0)
    plsc.subcore_barrier()

    def fire_loads(i, b):
        base = wid * EPW + i * C
        pltpu.async_copy(ef_hbm.at[pl.ds(base, C)], efd[b], lsem[b])
        pltpu.async_copy(tp_hbm.at[pl.ds(base, C)], tpd[b], lsem[b])

    def wait_loads(i, b):
        base = wid * EPW + i * C
        pltpu.make_async_copy(ef_hbm.at[pl.ds(base, C)], efd[b], lsem[b]).wait()
        pltpu.make_async_copy(tp_hbm.at[pl.ds(base, C)], tpd[b], lsem[b]).wait()

    def stage(i, b):
        @pl.when(i + 1 < NCHUNK)
        def _():
            fire_loads(i + 1, 1 - b)
        wait_loads(i, b)
        pltpu.sync_copy(efd[b], accf.at[idx2.at[i]], add=True)
        pltpu.sync_copy(tpd[b], acct.at[idx2.at[i]], add=True)

    fire_loads(0, 0)

    def outer(g, carry):
        stage(2 * g, 0)
        stage(2 * g + 1, 1)
        return carry
    lax.fori_loop(0, NCHUNK // 2, outer, 0)
    stage(NCHUNK - 1, 0)
    plsc.subcore_barrier()

    def wchunk(c, carry):
        base = pl.multiple_of(jnp.minimum(sid * 640 + c * C, N - C), 8)
        pltpu.sync_copy(accf.at[pl.ds(base, C)], ef_v0)
        pltpu.sync_copy(ef_v0, outf_hbm.at[cid, pl.ds(base, C)])
        pltpu.sync_copy(acct.at[pl.ds(base, C)], tp_v0)
        pltpu.sync_copy(tp_v0, outt_hbm.at[cid, pl.ds(base, C)])
        return carry
    lax.fori_loop(0, 8, wchunk, 0)


BE = 4000
BP = 2000


def _silu(x):
    return x * (1.0 / (1.0 + jnp.exp(-x)))


def _tc_prep_body(ns_ref, wa_ref, wb_ref, p_ref, q_ref):
    ns = ns_ref[...]
    p_ref[...] = jnp.dot(ns, wa_ref[...], preferred_element_type=jnp.float32)
    q_ref[...] = jnp.dot(ns, wb_ref[...], preferred_element_type=jnp.float32)


def _tc_prep(ns, wa, wb):
    grid = (N // BP,)
    full = lambda i: (0, 0)
    return pl.pallas_call(
        _tc_prep_body,
        grid=grid,
        in_specs=[
            pl.BlockSpec((BP, D), lambda i: (i, 0)),
            pl.BlockSpec((D, H), full),
            pl.BlockSpec((D, H), full),
        ],
        out_specs=[
            pl.BlockSpec((BP, H), lambda i: (i, 0)),
            pl.BlockSpec((BP, H), lambda i: (i, 0)),
        ],
        out_shape=[
            jax.ShapeDtypeStruct((N, H), jnp.float32),
            jax.ShapeDtypeStruct((N, H), jnp.float32),
        ],
    )(ns, wa, wb)


def _tc_edge_body(hpq_ref, cd_ref, ea_ref,
                  wr_ref, we_ref, be1_ref,
                  we2_ref, be2_ref, wc1_ref, bc1_ref, wc2t_ref,
                  ef_ref, tp_ref):
    cd = cd_ref[...]
    ea = ea_ref[...]
    rad = jnp.sum(cd * cd, axis=1, keepdims=True)
    pre = (hpq_ref[...]
           + jnp.dot(ea, we_ref[...], preferred_element_type=jnp.float32)
           + rad * wr_ref[...]
           + be1_ref[...])
    h = _silu(pre)
    ef = _silu(jnp.dot(h, we2_ref[...], preferred_element_type=jnp.float32) + be2_ref[...])
    ef_ref[...] = ef
    hc = _silu(jnp.dot(ef, wc1_ref[...], preferred_element_type=jnp.float32) + bc1_ref[...])
    cm = jnp.sum(hc * wc2t_ref[...], axis=1, keepdims=True)
    trans = jnp.clip(cm * cd, -100.0, 100.0)
    ones_col = (lax.broadcasted_iota(jnp.int32, (BE, 16), 1) == 3).astype(jnp.float32)
    tp_ref[...] = trans + ones_col


def _tc_edge(hpq, cdiff, edge_attr, wr, wee, be1, we2, be2, wc1, bc1, wc2t):
    grid = (E // BE,)
    full = lambda i: (0, 0)
    return pl.pallas_call(
        _tc_edge_body,
        grid=grid,
        in_specs=[
            pl.BlockSpec((BE, D), lambda i: (i, 0)),
            pl.BlockSpec((BE, 16), lambda i: (i, 0)),
            pl.BlockSpec((BE, EA), lambda i: (i, 0)),
            pl.BlockSpec((1, H), full),
            pl.BlockSpec((EA, H), full),
            pl.BlockSpec((1, H), full),
            pl.BlockSpec((H, H), full),
            pl.BlockSpec((1, H), full),
            pl.BlockSpec((H, H), full),
            pl.BlockSpec((1, H), full),
            pl.BlockSpec((1, H), full),
        ],
        out_specs=[
            pl.BlockSpec((BE, H), lambda i: (i, 0)),
            pl.BlockSpec((BE, 16), lambda i: (i, 0)),
        ],
        out_shape=[
            jax.ShapeDtypeStruct((E, H), jnp.float32),
            jax.ShapeDtypeStruct((E, 16), jnp.float32),
        ],
    )(hpq, cdiff, edge_attr, wr, wee, be1, we2, be2, wc1, bc1, wc2t)


BN = 2000


def _tc_node_body(ns_ref, cp_ref, vp_ref, sf_ref, st_ref,
                  wn1a_ref, wn1b_ref, bn1_ref, wn2_ref, bn2_ref,
                  wv1_ref, bv1_ref, wv2t_ref, bv2_ref,
                  nsout_ref, c3_ref):
    ns = ns_ref[...]
    sf = sf_ref[0] + sf_ref[1]
    st = st_ref[0] + st_ref[1]
    cnt = jnp.maximum(st[:, 3:4], 1.0)
    agg_f = sf / cnt
    agg_t = st / cnt
    pre = (jnp.dot(ns, wn1a_ref[...], preferred_element_type=jnp.float32)
           + jnp.dot(agg_f, wn1b_ref[...], preferred_element_type=jnp.float32)
           + bn1_ref[...])
    h = _silu(pre)
    nout = jnp.dot(h, wn2_ref[...], preferred_element_type=jnp.float32) + bn2_ref[...]
    nsout_ref[...] = ns + nout
    pv = jnp.dot(ns, wv1_ref[...], preferred_element_type=jnp.float32) + bv1_ref[...]
    hv = _silu(pv)
    cvm = jnp.sum(hv * wv2t_ref[...], axis=1, keepdims=True) + bv2_ref[...]
    mask3 = (lax.broadcasted_iota(jnp.int32, (BN, 16), 1) < 3).astype(jnp.float32)
    c3_ref[...] = cp_ref[...] + agg_t * mask3 + cvm * vp_ref[...]


def _tc_node(ns, coordp, velp, sumf, sumt, wn1a, wn1b, bn1, wn2, bn2, wv1, bv1, wv2t, bv2):
    grid = (N // BN,)
    full = lambda i: (0, 0)
    return pl.pallas_call(
        _tc_node_body,
        grid=grid,
        in_specs=[
            pl.BlockSpec((BN, D), lambda i: (i, 0)),
            pl.BlockSpec((BN, 16), lambda i: (i, 0)),
            pl.BlockSpec((BN, 16), lambda i: (i, 0)),
            pl.BlockSpec((NC, BN, H), lambda i: (0, i, 0)),
            pl.BlockSpec((NC, BN, 16), lambda i: (0, i, 0)),
            pl.BlockSpec((D, H), full),
            pl.BlockSpec((H, H), full),
            pl.BlockSpec((1, H), full),
            pl.BlockSpec((H, D), full),
            pl.BlockSpec((1, D), full),
            pl.BlockSpec((D, H), full),
            pl.BlockSpec((1, H), full),
            pl.BlockSpec((1, H), full),
            pl.BlockSpec((1, 1), full),
        ],
        out_specs=[
            pl.BlockSpec((BN, D), lambda i: (i, 0)),
            pl.BlockSpec((BN, 16), lambda i: (i, 0)),
        ],
        out_shape=[
            jax.ShapeDtypeStruct((N, D), jnp.float32),
            jax.ShapeDtypeStruct((N, 16), jnp.float32),
        ],
    )(ns, coordp, velp, sumf, sumt, wn1a, wn1b, bn1, wn2, bn2, wv1, bv1, wv2t, bv2)


def kernel(node_state, edge_index, coord, velocity, edge_attr,
           We1, be1, We2, be2, Wn1, bn1, Wn2, bn2,
           Wc1, bc1, Wc2, Wv1, bv1, Wv2, bv2):
    row = edge_index[0]
    col = edge_index[1]
    row3 = row.reshape(NW, NCHUNK, C)
    col3 = col.reshape(NW, NCHUNK, C)
    coordp = jnp.pad(coord, ((0, 0), (0, 13)))
    velp = jnp.pad(velocity, ((0, 0), (0, 13)))

    wa = We1[:D]
    wb = We1[D:2 * D]
    wr = We1[2 * D:2 * D + 1]
    wee = We1[2 * D + 1:]
    p_arr, q_arr = _tc_prep(node_state, wa, wb)
    hpq, cdiff = _make_sc_gather()(p_arr, q_arr, coordp, row3, col3)

    ef, tp = _tc_edge(hpq, cdiff, edge_attr,
                      wr, wee, be1.reshape(1, H),
                      We2, be2.reshape(1, H), Wc1, bc1.reshape(1, H),
                      Wc2.reshape(1, H))

    sumf, sumt = _make_sc_scatter()(row3, ef, tp)

    ns_new, c3p = _tc_node(node_state, coordp, velp, sumf, sumt,
                           Wn1[:D], Wn1[D:], bn1.reshape(1, H),
                           Wn2, bn2.reshape(1, D),
                           Wv1, bv1.reshape(1, H), Wv2.reshape(1, H),
                           bv2.reshape(1, 1))

    c3 = c3p[:, :3].reshape(N, 3, 1)
    v3 = velocity.reshape(N, 3, 1)
    return ns_new, c3, v3


# speedup vs baseline: 1.9348x; 1.0115x over previous
import functools

import jax
import jax.numpy as jnp
from jax import lax
from jax.experimental import pallas as pl
from jax.experimental.pallas import tpu as pltpu
from jax.experimental.pallas import tpu_sc as plsc

N = 10000
E = 320000
D = 128
EA = 16
H = 128

NC, NS = 2, 16
NW = NC * NS
EPW = E // NW
C = 80
NCHUNK = EPW // C
RPT = 632

@functools.cache
def _get_mesh():
    return plsc.VectorSubcoreMesh(
        core_axis_name="c", subcore_axis_name="s", num_cores=NC, num_subcores=NS)


@functools.cache
def _make_sc_gather():
    return functools.partial(
        pl.kernel,
        out_type=(
            jax.ShapeDtypeStruct((E, D), jnp.float32),
            jax.ShapeDtypeStruct((E, 16), jnp.float32),
        ),
        mesh=_get_mesh(),
        scratch_types=[
            pltpu.VMEM((NCHUNK, C), jnp.int32),
            pltpu.VMEM((NCHUNK, C), jnp.int32),
            pltpu.VMEM((C, D), jnp.float32),
            pltpu.VMEM((C, D), jnp.float32),
            pltpu.VMEM((C, 16), jnp.float32),
            pltpu.VMEM((C, 16), jnp.float32),
            pltpu.VMEM((C, D), jnp.float32),
            pltpu.VMEM((C, D), jnp.float32),
            pltpu.VMEM((C, 16), jnp.float32),
            pltpu.VMEM((C, 16), jnp.float32),
            pltpu.SemaphoreType.DMA,
            pltpu.SemaphoreType.DMA,
            pltpu.SemaphoreType.DMA,
            pltpu.SemaphoreType.DMA,
        ],
        compiler_params=pltpu.CompilerParams(use_tc_tiling_on_sc=False),
    )(_sc_gather_body)


def _sc_gather_body(p_hbm, q_hbm, coordp_hbm, row3_hbm, col3_hbm,
                    hpq_hbm, cdiff_hbm,
                    idx_r2, idx_c2,
                    src_v0, dst_v0, cr_v0, cc_v0,
                    src_v1, dst_v1, cr_v1, cc_v1,
                    gsem0, gsem1, wsem0, wsem1):
    wid = lax.axis_index("s") * NC + lax.axis_index("c")
    data = ((src_v0, dst_v0, cr_v0, cc_v0), (src_v1, dst_v1, cr_v1, cc_v1))
    gsem = (gsem0, gsem1)
    wsem = (wsem0, wsem1)

    pltpu.sync_copy(row3_hbm.at[wid], idx_r2)
    pltpu.sync_copy(col3_hbm.at[wid], idx_c2)

    def fire(i, b):
        srcb, dstb, crb, ccb = data[b]
        pltpu.async_copy(p_hbm.at[idx_r2.at[i]], srcb, gsem[b])
        pltpu.async_copy(q_hbm.at[idx_c2.at[i]], dstb, gsem[b])
        pltpu.async_copy(coordp_hbm.at[idx_r2.at[i]], crb, gsem[b])
        pltpu.async_copy(coordp_hbm.at[idx_c2.at[i]], ccb, gsem[b])

    def wait_gathers(i, b):
        srcb, dstb, crb, ccb = data[b]
        pltpu.make_async_copy(p_hbm.at[idx_r2.at[i]], srcb, gsem[b]).wait()
        pltpu.make_async_copy(q_hbm.at[idx_c2.at[i]], dstb, gsem[b]).wait()
        pltpu.make_async_copy(coordp_hbm.at[idx_r2.at[i]], crb, gsem[b]).wait()
        pltpu.make_async_copy(coordp_hbm.at[idx_c2.at[i]], ccb, gsem[b]).wait()

    def fire_writes(i, b):
        srcb, _, crb, _ = data[b]
        base = wid * EPW + i * C
        pltpu.async_copy(srcb, hpq_hbm.at[pl.ds(base, C)], wsem[b])
        pltpu.async_copy(crb, cdiff_hbm.at[pl.ds(base, C)], wsem[b])

    def drain_writes(i, b):
        srcb, _, crb, _ = data[b]
        base = wid * EPW + i * C
        pltpu.make_async_copy(srcb, hpq_hbm.at[pl.ds(base, C)], wsem[b]).wait()
        pltpu.make_async_copy(crb, cdiff_hbm.at[pl.ds(base, C)], wsem[b]).wait()

    def compute(b):
        srcb, dstb, crb, ccb = data[b]

        @plsc.parallel_loop(0, C, unroll=4)
        def rowop(j):
            for k in range(D // 16):
                sl = pl.ds(k * 16, 16)
                plsc.addupdate(srcb.at[j, sl], dstb[j, sl])
            plsc.addupdate(crb.at[j], -ccb[j])

    def stage(i, b):
        @pl.when(i + 1 < NCHUNK)
        def _():
            @pl.when(i >= 1)
            def __():
                drain_writes(i, 1 - b)
            fire(i + 1, 1 - b)
        wait_gathers(i, b)
        compute(b)
        fire_writes(i, b)

    fire(0, 0)

    def outer(g, carry):
        stage(2 * g, 0)
        stage(2 * g + 1, 1)
        return carry
    lax.fori_loop(0, NCHUNK // 2, outer, 0)
    stage(NCHUNK - 1, 0)
    drain_writes(NCHUNK - 1, 0)
    drain_writes(NCHUNK - 2, 1)


@functools.cache
def _make_sc_scatter():
    return functools.partial(
        pl.kernel,
        out_type=(
            jax.ShapeDtypeStruct((NC, N, H), jnp.float32),
            jax.ShapeDtypeStruct((NC, N, 16), jnp.float32),
        ),
        mesh=_get_mesh(),
        scratch_types=[
            pltpu.VMEM((NCHUNK, C), jnp.int32),
            pltpu.VMEM((C, H), jnp.float32),
            pltpu.VMEM((C, 16), jnp.float32),
            pltpu.VMEM((C, H), jnp.float32),
            pltpu.VMEM((C, 16), jnp.float32),
            pltpu.VMEM_SHARED((N, H), jnp.float32),
            pltpu.VMEM_SHARED((N, 16), jnp.float32),
            pltpu.SemaphoreType.DMA,
            pltpu.SemaphoreType.DMA,
        ],
        compiler_params=pltpu.CompilerParams(use_tc_tiling_on_sc=False),
    )(_sc_scatter_body)


def _sc_scatter_body(row3_hbm, ef_hbm, tp_hbm,
                     outf_hbm, outt_hbm,
                     idx2, ef_v0, tp_v0, ef_v1, tp_v1, accf, acct,
                     lsem0, lsem1):
    cid = lax.axis_index("c")
    sid = lax.axis_index("s")
    wid = sid * NC + cid
    efd = (ef_v0, ef_v1)
    tpd = (tp_v0, tp_v1)
    lsem = (lsem0, lsem1)

    pltpu.sync_copy(row3_hbm.at[wid], idx2)

    def zrow(j, carry):
        def zcol(k, c2):
            ef_v0[j, pl.ds(k * 16, 16)] = jnp.zeros((16,), jnp.float32)
            return c2
        lax.fori_loop(0, H // 16, zcol, 0)
        tp_v0[j] = jnp.zeros((16,), jnp.float32)
        return carry
    lax.fori_loop(0, C, zrow, 0)

    def zchunk(c, carry):
        base = pl.multiple_of(jnp.minimum(sid * 640 + c * C, N - C), 8)
        pltpu.sync_copy(ef_v0, accf.at[pl.ds(base, C)])
        pltpu.sync_copy(tp_v0, acct.at[pl.ds(base, C)])
        return carry
    lax.fori_loop(0, 8, zchunk, 0)
    plsc.subcore_barrier()

    def fire_loads(i, b):
        base = wid * EPW + i * C
        pltpu.async_copy(ef_hbm.at[pl.ds(base, C)], efd[b], lsem[b])
        pltpu.async_copy(tp_hbm.at[pl.ds(base, C)], tpd[b], lsem[b])

    def wait_loads(i, b):
        base = wid * EPW + i * C
        pltpu.make_async_copy(ef_hbm.at[pl.ds(base, C)], efd[b], lsem[b]).wait()
        pltpu.make_async_copy(tp_hbm.at[pl.ds(base, C)], tpd[b], lsem[b]).wait()

    def stage(i, b):
        @pl.when(i + 1 < NCHUNK)
        def _():
            fire_loads(i + 1, 1 - b)
        wait_loads(i, b)
        pltpu.sync_copy(efd[b], accf.at[idx2.at[i]], add=True)
        pltpu.sync_copy(tpd[b], acct.at[idx2.at[i]], add=True)

    fire_loads(0, 0)

    def outer(g, carry):
        stage(2 * g, 0)
        stage(2 * g + 1, 1)
        return carry
    lax.fori_loop(0, NCHUNK // 2, outer, 0)
    stage(NCHUNK - 1, 0)
    plsc.subcore_barrier()

    def wchunk(c, carry):
        base = pl.multiple_of(jnp.minimum(sid * 640 + c * C, N - C), 8)
        pltpu.sync_copy(accf.at[pl.ds(base, C)], ef_v0)
        pltpu.sync_copy(ef_v0, outf_hbm.at[cid, pl.ds(base, C)])
        pltpu.sync_copy(acct.at[pl.ds(base, C)], tp_v0)
        pltpu.sync_copy(tp_v0, outt_hbm.at[cid, pl.ds(base, C)])
        return carry
    lax.fori_loop(0, 8, wchunk, 0)


BE = 8000
BP = 2000


def _silu(x):
    return x * (1.0 / (1.0 + jnp.exp(-x)))


def _tc_prep_body(ns_ref, wa_ref, wb_ref, p_ref, q_ref):
    ns = ns_ref[...]
    p_ref[...] = jnp.dot(ns, wa_ref[...], preferred_element_type=jnp.float32)
    q_ref[...] = jnp.dot(ns, wb_ref[...], preferred_element_type=jnp.float32)


def _tc_prep(ns, wa, wb):
    grid = (N // BP,)
    full = lambda i: (0, 0)
    return pl.pallas_call(
        _tc_prep_body,
        grid=grid,
        in_specs=[
            pl.BlockSpec((BP, D), lambda i: (i, 0)),
            pl.BlockSpec((D, H), full),
            pl.BlockSpec((D, H), full),
        ],
        out_specs=[
            pl.BlockSpec((BP, H), lambda i: (i, 0)),
            pl.BlockSpec((BP, H), lambda i: (i, 0)),
        ],
        out_shape=[
            jax.ShapeDtypeStruct((N, H), jnp.float32),
            jax.ShapeDtypeStruct((N, H), jnp.float32),
        ],
    )(ns, wa, wb)


def _tc_edge_body(hpq_ref, cd_ref, ea_ref,
                  wr_ref, we_ref, be1_ref,
                  we2_ref, be2_ref, wc1_ref, bc1_ref, wc2t_ref,
                  ef_ref, tp_ref):
    cd = cd_ref[...]
    ea = ea_ref[...]
    rad = jnp.sum(cd * cd, axis=1, keepdims=True)
    pre = (hpq_ref[...]
           + jnp.dot(ea, we_ref[...], preferred_element_type=jnp.float32)
           + rad * wr_ref[...]
           + be1_ref[...])
    h = _silu(pre)
    ef = _silu(jnp.dot(h, we2_ref[...], preferred_element_type=jnp.float32) + be2_ref[...])
    ef_ref[...] = ef
    hc = _silu(jnp.dot(ef, wc1_ref[...], preferred_element_type=jnp.float32) + bc1_ref[...])
    cm = jnp.sum(hc * wc2t_ref[...], axis=1, keepdims=True)
    trans = jnp.clip(cm * cd, -100.0, 100.0)
    ones_col = (lax.broadcasted_iota(jnp.int32, (BE, 16), 1) == 3).astype(jnp.float32)
    tp_ref[...] = trans + ones_col


def _tc_edge(hpq, cdiff, edge_attr, wr, wee, be1, we2, be2, wc1, bc1, wc2t):
    grid = (E // BE,)
    full = lambda i: (0, 0)
    return pl.pallas_call(
        _tc_edge_body,
        grid=grid,
        in_specs=[
            pl.BlockSpec((BE, D), lambda i: (i, 0)),
            pl.BlockSpec((BE, 16), lambda i: (i, 0)),
            pl.BlockSpec((BE, EA), lambda i: (i, 0)),
            pl.BlockSpec((1, H), full),
            pl.BlockSpec((EA, H), full),
            pl.BlockSpec((1, H), full),
            pl.BlockSpec((H, H), full),
            pl.BlockSpec((1, H), full),
            pl.BlockSpec((H, H), full),
            pl.BlockSpec((1, H), full),
            pl.BlockSpec((1, H), full),
        ],
        out_specs=[
            pl.BlockSpec((BE, H), lambda i: (i, 0)),
            pl.BlockSpec((BE, 16), lambda i: (i, 0)),
        ],
        out_shape=[
            jax.ShapeDtypeStruct((E, H), jnp.float32),
            jax.ShapeDtypeStruct((E, 16), jnp.float32),
        ],
    )(hpq, cdiff, edge_attr, wr, wee, be1, we2, be2, wc1, bc1, wc2t)


BN = 2000


def _tc_node_body(ns_ref, cp_ref, vp_ref, sf_ref, st_ref,
                  wn1a_ref, wn1b_ref, bn1_ref, wn2_ref, bn2_ref,
                  wv1_ref, bv1_ref, wv2t_ref, bv2_ref,
                  nsout_ref, c3_ref):
    ns = ns_ref[...]
    sf = sf_ref[0] + sf_ref[1]
    st = st_ref[0] + st_ref[1]
    cnt = jnp.maximum(st[:, 3:4], 1.0)
    agg_f = sf / cnt
    agg_t = st / cnt
    pre = (jnp.dot(ns, wn1a_ref[...], preferred_element_type=jnp.float32)
           + jnp.dot(agg_f, wn1b_ref[...], preferred_element_type=jnp.float32)
           + bn1_ref[...])
    h = _silu(pre)
    nout = jnp.dot(h, wn2_ref[...], preferred_element_type=jnp.float32) + bn2_ref[...]
    nsout_ref[...] = ns + nout
    pv = jnp.dot(ns, wv1_ref[...], preferred_element_type=jnp.float32) + bv1_ref[...]
    hv = _silu(pv)
    cvm = jnp.sum(hv * wv2t_ref[...], axis=1, keepdims=True) + bv2_ref[...]
    mask3 = (lax.broadcasted_iota(jnp.int32, (BN, 16), 1) < 3).astype(jnp.float32)
    c3_ref[...] = cp_ref[...] + agg_t * mask3 + cvm * vp_ref[...]


def _tc_node(ns, coordp, velp, sumf, sumt, wn1a, wn1b, bn1, wn2, bn2, wv1, bv1, wv2t, bv2):
    grid = (N // BN,)
    full = lambda i: (0, 0)
    return pl.pallas_call(
        _tc_node_body,
        grid=grid,
        in_specs=[
            pl.BlockSpec((BN, D), lambda i: (i, 0)),
            pl.BlockSpec((BN, 16), lambda i: (i, 0)),
            pl.BlockSpec((BN, 16), lambda i: (i, 0)),
            pl.BlockSpec((NC, BN, H), lambda i: (0, i, 0)),
            pl.BlockSpec((NC, BN, 16), lambda i: (0, i, 0)),
            pl.BlockSpec((D, H), full),
            pl.BlockSpec((H, H), full),
            pl.BlockSpec((1, H), full),
            pl.BlockSpec((H, D), full),
            pl.BlockSpec((1, D), full),
            pl.BlockSpec((D, H), full),
            pl.BlockSpec((1, H), full),
            pl.BlockSpec((1, H), full),
            pl.BlockSpec((1, 1), full),
        ],
        out_specs=[
            pl.BlockSpec((BN, D), lambda i: (i, 0)),
            pl.BlockSpec((BN, 16), lambda i: (i, 0)),
        ],
        out_shape=[
            jax.ShapeDtypeStruct((N, D), jnp.float32),
            jax.ShapeDtypeStruct((N, 16), jnp.float32),
        ],
    )(ns, coordp, velp, sumf, sumt, wn1a, wn1b, bn1, wn2, bn2, wv1, bv1, wv2t, bv2)


def kernel(node_state, edge_index, coord, velocity, edge_attr,
           We1, be1, We2, be2, Wn1, bn1, Wn2, bn2,
           Wc1, bc1, Wc2, Wv1, bv1, Wv2, bv2):
    row = edge_index[0]
    col = edge_index[1]
    row3 = row.reshape(NW, NCHUNK, C)
    col3 = col.reshape(NW, NCHUNK, C)
    coordp = jnp.pad(coord, ((0, 0), (0, 13)))
    velp = jnp.pad(velocity, ((0, 0), (0, 13)))

    wa = We1[:D]
    wb = We1[D:2 * D]
    wr = We1[2 * D:2 * D + 1]
    wee = We1[2 * D + 1:]
    p_arr, q_arr = _tc_prep(node_state, wa, wb)
    hpq, cdiff = _make_sc_gather()(p_arr, q_arr, coordp, row3, col3)

    ef, tp = _tc_edge(hpq, cdiff, edge_attr,
                      wr, wee, be1.reshape(1, H),
                      We2, be2.reshape(1, H), Wc1, bc1.reshape(1, H),
                      Wc2.reshape(1, H))

    sumf, sumt = _make_sc_scatter()(row3, ef, tp)

    ns_new, c3p = _tc_node(node_state, coordp, velp, sumf, sumt,
                           Wn1[:D], Wn1[D:], bn1.reshape(1, H),
                           Wn2, bn2.reshape(1, D),
                           Wv1, bv1.reshape(1, H), Wv2.reshape(1, H),
                           bv2.reshape(1, 1))

    c3 = c3p[:, :3].reshape(N, 3, 1)
    v3 = velocity.reshape(N, 3, 1)
    return ns_new, c3, v3
